# Initial kernel scaffold; baseline (speedup 1.0000x reference)
import jax
import jax.numpy as jnp
from jax.experimental import pallas as pl


def kernel(lstm_out, first_notes, node_type, leaf_token, parent_idx, is_leaf, is_ptr, tree_id, ptr_pos, root_idx, embedding, leaf_W1, leaf_b1, leaf_W2, leaf_b2, node_W, node_b, ptr_W, ptr_b, ff_W1, ff_b1, ff_W2, ff_b2, tail_W, tail_b):
    raise NotImplementedError("write your pallas kernel here")


# speedup vs baseline: 4.7267x; 4.7267x over previous
import functools

import jax
import jax.numpy as jnp
from jax import lax
from jax.experimental import pallas as pl
from jax.experimental.pallas import tpu as pltpu
from jax.experimental.pallas import tpu_sc as plsc

N = 131072
B = 256
T = 256
HID = 32
ED = 16
DICT = 200
TPAD = 208
LO_ROWS = B * T
LO_SPREAD = 2048
LO_PAD = LO_ROWS + LO_SPREAD
BIG = 1e30

NC = 2
NS = 16
NW = NC * NS
NPW = N // NW
GK = 1024
SK = 2048
PASS = 44032
NPASS = 3
TRASH = 512

_f32 = jnp.float32
_i32 = jnp.int32


def _sds(shape, dtype=_f32):
    return jax.ShapeDtypeStruct(shape, dtype)


def _fold_body(emb_ref, lw1a_ref, lw1b_ref, lw2_ref, lb1_ref, lb2_ref,
               nwp_ref, nwn_ref, nb_ref, fn_ref, pwf_ref, pb_ref, wc_ref,
               tn_ref, tl1_ref, tl2_ref, tp_ref, fnh_ref,
               w11_ref, w21_ref, w12_ref, w22_ref):
    emb = emb_ref[...]
    sel = (lax.broadcasted_iota(_i32, (128, 16), 0) % 16 ==
           lax.broadcasted_iota(_i32, (128, 16), 1)).astype(_f32)
    selt = (lax.broadcasted_iota(_i32, (16, 128), 0) ==
            lax.broadcasted_iota(_i32, (16, 128), 1) % 16).astype(_f32)
    blk = (lax.broadcasted_iota(_i32, (128, 128), 0) // 16 ==
           lax.broadcasted_iota(_i32, (128, 128), 1) // 16).astype(_f32)
    wc = wc_ref[...]

    def packw(w16):
        t = jnp.dot(jnp.dot(sel, w16, preferred_element_type=_f32), selt,
                    preferred_element_type=_f32)
        return t * blk

    w11_ref[...] = packw(wc[:16, :16])
    w21_ref[...] = packw(wc[16:, :16])
    w12_ref[...] = packw(wc[:16, 16:])
    w22_ref[...] = packw(wc[16:, 16:])
    zpad = jnp.zeros((TPAD - DICT, HID), _f32)
    tn = jnp.dot(emb, nwn_ref[...], preferred_element_type=_f32)
    tn_ref[...] = jnp.concatenate([tn, zpad], axis=0)
    a1 = jnp.dot(lw1a_ref[...], lw2_ref[...], preferred_element_type=_f32)
    tl1 = jnp.dot(emb, a1, preferred_element_type=_f32)
    tl1_ref[...] = jnp.concatenate([tl1, zpad], axis=0)
    a2 = jnp.dot(lw1b_ref[...], lw2_ref[...], preferred_element_type=_f32)
    cl = jnp.dot(lb1_ref[...], lw2_ref[...], preferred_element_type=_f32) + lb2_ref[...]
    tl2 = jnp.dot(emb, a2, preferred_element_type=_f32) + cl
    tl2_ref[...] = jnp.concatenate([tl2, zpad], axis=0)
    tp = jnp.dot(emb, nwp_ref[...], preferred_element_type=_f32) + nb_ref[...]
    ridx = lax.broadcasted_iota(_i32, (TPAD - DICT, HID), 0)
    neg = jnp.where(ridx == 1, _f32(-BIG), _f32(0.0))
    tp_ref[...] = jnp.concatenate([tp, neg], axis=0)
    fnh_ref[...] = jnp.dot(fn_ref[...], pwf_ref[...], preferred_element_type=_f32) + pb_ref[...]


def _fold_call(emb, lw1a, lw1b, lw2, lb1, lb2, nwp, nwn, nb, fn, pwf, pb, wc):
    return pl.pallas_call(
        _fold_body,
        out_shape=[_sds((TPAD, HID)), _sds((TPAD, HID)), _sds((TPAD, HID)),
                   _sds((TPAD, HID)), _sds((B, HID)),
                   _sds((128, 128)), _sds((128, 128)), _sds((128, 128)),
                   _sds((128, 128))],
    )(emb, lw1a, lw1b, lw2, lb1, lb2, nwp, nwn, nb, fn, pwf, pb, wc)


_LO_TREES = 8
_LO_BL = _LO_TREES * T
_LO_STEPS = LO_PAD // _LO_BL


def _lo2_body(lstm_ref, fnh_ref, pwl_ref, out_ref):
    i = pl.program_id(0)
    x = lstm_ref[...].reshape(_LO_BL, 128)
    y = jnp.dot(x, pwl_ref[...], preferred_element_type=_f32)
    y = (y.reshape(_LO_TREES, T, HID) + fnh_ref[...][:, None, :]).reshape(_LO_BL, HID)
    out_ref[...] = jnp.where(i >= LO_ROWS // _LO_BL, _f32(0.0), y)


def _lo2_call(lstm_out, fnh, pwl):
    return pl.pallas_call(
        _lo2_body,
        grid=(_LO_STEPS,),
        in_specs=[
            pl.BlockSpec((_LO_TREES, T, 128), lambda i: (jnp.minimum(i, B // _LO_TREES - 1), 0, 0)),
            pl.BlockSpec((_LO_TREES, HID), lambda i: (jnp.minimum(i, B // _LO_TREES - 1), 0)),
            pl.BlockSpec((128, HID), lambda i: (0, 0)),
        ],
        out_specs=pl.BlockSpec((_LO_BL, HID), lambda i: (i, 0)),
        out_shape=_sds((LO_PAD, HID)),
    )(lstm_out, fnh, pwl)


_SC_MESH = plsc.VectorSubcoreMesh(core_axis_name="c", subcore_axis_name="s")


@functools.partial(
    pl.kernel,
    out_type=[_sds((N, 16)), _sds((N, 16)), _sds((N, 16)), _sds((N, 16))],
    mesh=_SC_MESH,
    compiler_params=pltpu.CompilerParams(needs_layout_passes=False, use_tc_tiling_on_sc=False),
    scratch_types=[
        pltpu.VMEM((TPAD, HID), _f32),
        pltpu.VMEM((TPAD, HID), _f32),
        pltpu.VMEM((TPAD, HID), _f32),
        pltpu.VMEM((TPAD, HID), _f32),
        pltpu.VMEM((GK,), _i32),
        pltpu.VMEM((GK,), _i32),
        pltpu.VMEM((GK,), _i32),
        pltpu.VMEM((GK,), _i32),
        pltpu.VMEM((GK,), _i32),
        pltpu.VMEM((GK,), _i32),
        pltpu.VMEM((GK // 128, 128), _i32),
        pltpu.VMEM((GK // 128, 128), _i32),
        pltpu.VMEM((GK // 128, 128), _i32),
        pltpu.VMEM((GK, HID), _f32),
        pltpu.VMEM((GK, HID), _f32),
        pltpu.SemaphoreType.DMA,
    ],
)
def _gather_kernel(nt_hbm, lt_hbm, pidx2_hbm, tid_hbm, pp_hbm, il_hbm, ip_hbm,
                   nt1_hbm, tn_hbm, tl1_hbm, tl2_hbm, tp_hbm, lo2_hbm,
                   base_lo_hbm, base_hi_hbm, pre_lo_hbm, pre_hi_hbm,
                   tn_v, tl1_v, tl2_v, tp_v,
                   nt_v, lt_v, tid_v, pp_v, il_v, ip_v,
                   pidx_v, ntp_v, flat_v, lo_v, pre_v, sem):
    wid = lax.axis_index("s") * NC + lax.axis_index("c")
    pltpu.sync_copy(tn_hbm, tn_v)
    pltpu.sync_copy(tl1_hbm, tl1_v)
    pltpu.sync_copy(tl2_hbm, tl2_v)
    pltpu.sync_copy(tp_hbm, tp_v)
    iota16 = lax.iota(_i32, 16)
    for chunk in range(NPW // GK):
        node0 = pl.multiple_of(wid * NPW + chunk * GK, GK)
        cps = [
            pltpu.async_copy(nt_hbm.at[pl.ds(node0, GK)], nt_v, sem),
            pltpu.async_copy(lt_hbm.at[pl.ds(node0, GK)], lt_v, sem),
            pltpu.async_copy(tid_hbm.at[pl.ds(node0, GK)], tid_v, sem),
            pltpu.async_copy(pp_hbm.at[pl.ds(node0, GK)], pp_v, sem),
            pltpu.async_copy(il_hbm.at[pl.ds(node0, GK)], il_v, sem),
            pltpu.async_copy(ip_hbm.at[pl.ds(node0, GK)], ip_v, sem),
            pltpu.async_copy(pidx2_hbm.at[pl.ds(pl.multiple_of(node0 // 128, 8), GK // 128)], pidx_v, sem),
        ]
        for cp in cps:
            cp.wait()

        def pass_a(g, carry):
            r = g // 8
            s0 = (g % 8) * 16
            tid = tid_v[pl.ds(g * 16, 16)]
            pp = pp_v[pl.ds(g * 16, 16)]
            il = il_v[pl.ds(g * 16, 16)]
            ip = ip_v[pl.ds(g * 16, 16)]
            mpb = il * ip
            dump = LO_ROWS + ((g * 16) % LO_SPREAD) + iota16
            flat = jnp.where(mpb == 1, tid * T + pp, dump)
            flat_v[r, pl.ds(s0, 16)] = flat
            return carry

        lax.fori_loop(0, GK // 16, pass_a, 0)

        gcps = []
        for j in range(GK // 128):
            gcps.append(pltpu.async_copy(nt1_hbm.at[pidx_v.at[j]],
                                         ntp_v.at[j], sem))
            gcps.append(pltpu.async_copy(lo2_hbm.at[flat_v.at[j]],
                                         lo_v.at[pl.ds(j * 128, 128)], sem))
        for cp in gcps:
            cp.wait()

        def pass_b(g, carry):
            r = g // 8
            s0 = (g % 8) * 16
            nt = nt_v[pl.ds(g * 16, 16)]
            lt = lt_v[pl.ds(g * 16, 16)]
            il = il_v[pl.ds(g * 16, 16)]
            ip = ip_v[pl.ds(g * 16, 16)]
            ntp = ntp_v[r, pl.ds(s0, 16)]
            mlb = il * (1 - ip)
            eff_l1 = jnp.where(mlb == 1, nt, DICT)
            eff_l2 = jnp.where(mlb == 1, lt, DICT)
            eff_p = jnp.where(il == 1, DICT + 1, ntp)
            rows = g * 16 + iota16
            for c in range(HID):
                cvec = jnp.full((16,), c, _i32)
                v_l = (plsc.load_gather(tl1_v, [eff_l1, cvec]) +
                       plsc.load_gather(tl2_v, [eff_l2, cvec]))
                plsc.addupdate_scatter(lo_v, [rows, cvec], v_l)
                v_p = (plsc.load_gather(tn_v, [nt, cvec]) +
                       plsc.load_gather(tp_v, [eff_p, cvec]))
                plsc.store_scatter(pre_v, [rows, cvec], v_p)
            return carry

        lax.fori_loop(0, GK // 16, pass_b, 0)

        pltpu.sync_copy(lo_v.at[:, pl.ds(0, 16)], base_lo_hbm.at[pl.ds(node0, GK)])
        pltpu.sync_copy(lo_v.at[:, pl.ds(16, 16)], base_hi_hbm.at[pl.ds(node0, GK)])
        pltpu.sync_copy(pre_v.at[:, pl.ds(0, 16)], pre_lo_hbm.at[pl.ds(node0, GK)])
        pltpu.sync_copy(pre_v.at[:, pl.ds(16, 16)], pre_hi_hbm.at[pl.ds(node0, GK)])


@functools.partial(
    pl.kernel,
    out_type=[_sds((N, 16)), _sds((N, 16))],
    mesh=_SC_MESH,
    compiler_params=pltpu.CompilerParams(needs_layout_passes=False, use_tc_tiling_on_sc=False),
    scratch_types=[
        pltpu.VMEM((SK, 16), _f32),
        pltpu.VMEM((SK // 128, 128), _i32),
        pltpu.VMEM((SK // 128, 128), _i32),
        pltpu.VMEM((SK, 16), _f32),
        pltpu.VMEM_SHARED((PASS + TRASH, 16), _f32),
        pltpu.SemaphoreType.DMA,
    ],
)
def _scatter_kernel(h_lo_hbm, h_hi_hbm, pidx2_hbm, cs_lo_hbm, cs_hi_hbm,
                    upd_v, pidx_v, eff_v, zb_v, acc, sem):
    cid = lax.axis_index("c")
    sid = lax.axis_index("s")
    z16 = jnp.zeros((16,), _f32)

    def zero_body(i, carry):
        zb_v[i, :] = z16
        return carry

    lax.fori_loop(0, SK, zero_body, 0)

    def run_core(h_hbm, out_hbm):
        for p in range(NPASS):
            lo_bound = p * PASS
            valid = min(PASS, N - p * PASS)
            rpt = (PASS + TRASH) // NS
            r0 = pl.multiple_of(sid * rpt, 8)
            pltpu.sync_copy(zb_v, acc.at[pl.ds(r0, SK)])
            pltpu.sync_copy(zb_v.at[pl.ds(0, rpt - SK)],
                            acc.at[pl.ds(r0 + SK, rpt - SK)])
            plsc.subcore_barrier()
            for chunk in range(N // NS // SK):
                src0 = pl.multiple_of(sid * (N // NS) + chunk * SK, SK)
                cp1 = pltpu.async_copy(h_hbm.at[pl.ds(src0, SK)], upd_v, sem)
                cp2 = pltpu.async_copy(pidx2_hbm.at[pl.ds(pl.multiple_of(src0 // 128, 8), SK // 128)],
                                       pidx_v, sem)
                cp1.wait()
                cp2.wait()

                def eff_body(g, carry):
                    r = g // 8
                    s0 = (g % 8) * 16
                    x = pidx_v[r, pl.ds(s0, 16)]
                    inb = jnp.logical_and(x >= lo_bound, x < lo_bound + valid)
                    e = jnp.where(inb, x - lo_bound,
                                  PASS + jnp.bitwise_and(x, TRASH - 1))
                    eff_v[r, pl.ds(s0, 16)] = e
                    return carry

                lax.fori_loop(0, SK // 16, eff_body, 0)
                for j in range(SK // 128):
                    pltpu.sync_copy(upd_v.at[pl.ds(j * 128, 128)],
                                    acc.at[eff_v.at[j]], add=True)
            plsc.subcore_barrier()
            wb = valid // NS
            wb0 = pl.multiple_of(sid * wb, 8)
            pltpu.sync_copy(acc.at[pl.ds(wb0, wb)],
                            out_hbm.at[pl.ds(lo_bound + wb0, wb)])
            plsc.subcore_barrier()

    @pl.when(cid == 0)
    def _():
        run_core(h_lo_hbm, cs_lo_hbm)

    @pl.when(cid == 1)
    def _():
        run_core(h_hi_hbm, cs_hi_hbm)


@functools.partial(
    pl.kernel,
    out_type=[_sds((B, 16)), _sds((B, 16))],
    mesh=_SC_MESH,
    compiler_params=pltpu.CompilerParams(needs_layout_passes=False, use_tc_tiling_on_sc=False),
    scratch_types=[
        pltpu.VMEM((B // 128, 128), _i32),
        pltpu.VMEM((B, 16), _f32),
        pltpu.VMEM((B, 16), _f32),
        pltpu.SemaphoreType.DMA,
    ],
)
def _root_kernel(ridx2_hbm, h_lo_hbm, h_hi_hbm, g_lo_hbm, g_hi_hbm,
                 idx_v, lo_v, hi_v, sem):
    wid = lax.axis_index("s") * NC + lax.axis_index("c")

    @pl.when(wid == 0)
    def _():
        pltpu.sync_copy(ridx2_hbm, idx_v)
        for j in range(B // 128):
            pltpu.async_copy(h_lo_hbm.at[idx_v.at[j]],
                             lo_v.at[pl.ds(j * 128, 128)], sem).wait()
            pltpu.async_copy(h_hi_hbm.at[idx_v.at[j]],
                             hi_v.at[pl.ds(j * 128, 128)], sem).wait()
        pltpu.sync_copy(lo_v, g_lo_hbm)
        pltpu.sync_copy(hi_v, g_hi_hbm)


_UP_ROWS = N // 8
_UP_BL = 2048


def _update_body(cs_lo_ref, cs_hi_ref, base_lo_ref, base_hi_ref,
                 pre_lo_ref, pre_hi_ref, w11_ref, w21_ref, w12_ref, w22_ref,
                 h_lo_ref, h_hi_ref):
    cl = cs_lo_ref[...]
    ch = cs_hi_ref[...]
    t_lo = jnp.maximum(
        pre_lo_ref[...] +
        jnp.dot(cl, w11_ref[...], preferred_element_type=_f32) +
        jnp.dot(ch, w21_ref[...], preferred_element_type=_f32), _f32(0.0))
    t_hi = jnp.maximum(
        pre_hi_ref[...] +
        jnp.dot(cl, w12_ref[...], preferred_element_type=_f32) +
        jnp.dot(ch, w22_ref[...], preferred_element_type=_f32), _f32(0.0))
    h_lo_ref[...] = base_lo_ref[...] + t_lo
    h_hi_ref[...] = base_hi_ref[...] + t_hi


def _update_call(cs_lo, cs_hi, base_lo8, base_hi8, pre_lo8, pre_hi8,
                 w11, w21, w12, w22):
    bspec = pl.BlockSpec((_UP_BL, 128), lambda i: (i, 0))
    wspec = pl.BlockSpec((128, 128), lambda i: (0, 0))
    h_lo8, h_hi8 = pl.pallas_call(
        _update_body,
        grid=(_UP_ROWS // _UP_BL,),
        in_specs=[bspec, bspec, bspec, bspec, bspec, bspec,
                  wspec, wspec, wspec, wspec],
        out_specs=[bspec, bspec],
        out_shape=[_sds((_UP_ROWS, 128)), _sds((_UP_ROWS, 128))],
    )(cs_lo.reshape(_UP_ROWS, 128), cs_hi.reshape(_UP_ROWS, 128),
      base_lo8, base_hi8, pre_lo8, pre_hi8, w11, w21, w12, w22)
    return h_lo8.reshape(N, 16), h_hi8.reshape(N, 16)


def _head_body(g_lo_ref, g_hi_ref, w1_ref, b1_ref, w2_ref, b2_ref,
               tw_ref, tb_ref, out_ref):
    g = jnp.concatenate([g_lo_ref[...], g_hi_ref[...]], axis=1)
    t = jnp.dot(g, w1_ref[...], preferred_element_type=_f32) + b1_ref[...]
    t = jnp.dot(t, w2_ref[...], preferred_element_type=_f32) + b2_ref[...]
    out_ref[...] = jnp.dot(t, tw_ref[...], preferred_element_type=_f32) + tb_ref[...]


def _head_call(g_lo, g_hi, w1, b1, w2, b2, tw, tb):
    return pl.pallas_call(
        _head_body,
        out_shape=_sds((B, 1)),
    )(g_lo, g_hi, w1, b1, w2, b2, tw, tb)


def kernel(lstm_out, first_notes, node_type, leaf_token, parent_idx, is_leaf,
           is_ptr, tree_id, ptr_pos, root_idx, embedding, leaf_W1, leaf_b1,
           leaf_W2, leaf_b2, node_W, node_b, ptr_W, ptr_b, ff_W1, ff_b1,
           ff_W2, ff_b2, tail_W, tail_b):
    nt = node_type.astype(_i32)
    lt = leaf_token.astype(_i32)
    pidx = parent_idx.astype(_i32)
    tid = tree_id.astype(_i32)
    pp = ptr_pos.astype(_i32)
    il = is_leaf.astype(_i32)
    ip = is_ptr.astype(_i32)

    t_n, t_l1, t_l2, t_p, fnh, w11, w21, w12, w22 = _fold_call(
        embedding, leaf_W1[:ED], leaf_W1[ED:], leaf_W2,
        leaf_b1.reshape(1, HID), leaf_b2.reshape(1, HID),
        node_W[:ED], node_W[ED:2 * ED], node_b.reshape(1, HID),
        first_notes, ptr_W[:64], ptr_b.reshape(1, HID), node_W[2 * ED:])
    lo2 = _lo2_call(lstm_out, fnh, ptr_W[64:])

    pidx2 = pidx.reshape(N // 128, 128)
    base_lo, base_hi, pre_lo, pre_hi = _gather_kernel(
        nt, lt, pidx2, tid, pp, il, ip, nt, t_n, t_l1, t_l2, t_p, lo2)

    base_lo8 = base_lo.reshape(_UP_ROWS, 128)
    base_hi8 = base_hi.reshape(_UP_ROWS, 128)
    pre_lo8 = pre_lo.reshape(_UP_ROWS, 128)
    pre_hi8 = pre_hi.reshape(_UP_ROWS, 128)
    h_lo, h_hi = base_lo, base_hi
    for _ in range(8):
        cs_lo, cs_hi = _scatter_kernel(h_lo, h_hi, pidx2)
        h_lo, h_hi = _update_call(cs_lo, cs_hi, base_lo8, base_hi8,
                                  pre_lo8, pre_hi8, w11, w21, w12, w22)

    g_lo, g_hi = _root_kernel(root_idx.astype(_i32).reshape(B // 128, 128),
                              h_lo, h_hi)
    return _head_call(g_lo, g_hi, ff_W1, ff_b1.reshape(1, HID),
                      ff_W2, ff_b2.reshape(1, HID), tail_W,
                      tail_b.reshape(1, 1))


# speedup vs baseline: 4.9139x; 1.0396x over previous
import functools

import jax
import jax.numpy as jnp
from jax import lax
from jax.experimental import pallas as pl
from jax.experimental.pallas import tpu as pltpu
from jax.experimental.pallas import tpu_sc as plsc

N = 131072
B = 256
T = 256
HID = 32
ED = 16
DICT = 200
TPAD = 208
LO_ROWS = B * T
LO_SPREAD = 2048
LO_PAD = LO_ROWS + LO_SPREAD
BIG = 1e30

NC = 2
NS = 16
NW = NC * NS
NPW = N // NW
GK = 1024
SK = 2048
PASS = 44032
NPASS = 3
TRASH = 512

_f32 = jnp.float32
_i32 = jnp.int32


def _sds(shape, dtype=_f32):
    return jax.ShapeDtypeStruct(shape, dtype)


def _fold_body(emb_ref, lw1a_ref, lw1b_ref, lw2_ref, lb1_ref, lb2_ref,
               nwp_ref, nwn_ref, nb_ref, fn_ref, pwf_ref, pb_ref, wc_ref,
               tn_ref, tl1_ref, tl2_ref, tp_ref, fnh_ref,
               w11_ref, w21_ref, w12_ref, w22_ref):
    emb = emb_ref[...]
    sel = (lax.broadcasted_iota(_i32, (128, 16), 0) % 16 ==
           lax.broadcasted_iota(_i32, (128, 16), 1)).astype(_f32)
    selt = (lax.broadcasted_iota(_i32, (16, 128), 0) ==
            lax.broadcasted_iota(_i32, (16, 128), 1) % 16).astype(_f32)
    blk = (lax.broadcasted_iota(_i32, (128, 128), 0) // 16 ==
           lax.broadcasted_iota(_i32, (128, 128), 1) // 16).astype(_f32)
    wc = wc_ref[...]

    def packw(w16):
        t = jnp.dot(jnp.dot(sel, w16, preferred_element_type=_f32), selt,
                    preferred_element_type=_f32)
        return t * blk

    w11_ref[...] = packw(wc[:16, :16])
    w21_ref[...] = packw(wc[16:, :16])
    w12_ref[...] = packw(wc[:16, 16:])
    w22_ref[...] = packw(wc[16:, 16:])
    zpad = jnp.zeros((TPAD - DICT, HID), _f32)
    tn = jnp.dot(emb, nwn_ref[...], preferred_element_type=_f32)
    tn_ref[...] = jnp.concatenate([tn, zpad], axis=0)
    a1 = jnp.dot(lw1a_ref[...], lw2_ref[...], preferred_element_type=_f32)
    tl1 = jnp.dot(emb, a1, preferred_element_type=_f32)
    tl1_ref[...] = jnp.concatenate([tl1, zpad], axis=0)
    a2 = jnp.dot(lw1b_ref[...], lw2_ref[...], preferred_element_type=_f32)
    cl = jnp.dot(lb1_ref[...], lw2_ref[...], preferred_element_type=_f32) + lb2_ref[...]
    tl2 = jnp.dot(emb, a2, preferred_element_type=_f32) + cl
    tl2_ref[...] = jnp.concatenate([tl2, zpad], axis=0)
    tp = jnp.dot(emb, nwp_ref[...], preferred_element_type=_f32) + nb_ref[...]
    ridx = lax.broadcasted_iota(_i32, (TPAD - DICT, HID), 0)
    neg = jnp.where(ridx == 1, _f32(-BIG), _f32(0.0))
    tp_ref[...] = jnp.concatenate([tp, neg], axis=0)
    fnh_ref[...] = jnp.dot(fn_ref[...], pwf_ref[...], preferred_element_type=_f32) + pb_ref[...]


def _fold_call(emb, lw1a, lw1b, lw2, lb1, lb2, nwp, nwn, nb, fn, pwf, pb, wc):
    return pl.pallas_call(
        _fold_body,
        out_shape=[_sds((TPAD, HID)), _sds((TPAD, HID)), _sds((TPAD, HID)),
                   _sds((TPAD, HID)), _sds((B, HID)),
                   _sds((128, 128)), _sds((128, 128)), _sds((128, 128)),
                   _sds((128, 128))],
    )(emb, lw1a, lw1b, lw2, lb1, lb2, nwp, nwn, nb, fn, pwf, pb, wc)


_LO_TREES = 8
_LO_BL = _LO_TREES * T
_LO_STEPS = LO_PAD // _LO_BL


def _lo2_body(lstm_ref, fnh_ref, pwl_ref, out_ref):
    i = pl.program_id(0)
    x = lstm_ref[...].reshape(_LO_BL, 128)
    y = jnp.dot(x, pwl_ref[...], preferred_element_type=_f32)
    y = (y.reshape(_LO_TREES, T, HID) + fnh_ref[...][:, None, :]).reshape(_LO_BL, HID)
    out_ref[...] = jnp.where(i >= LO_ROWS // _LO_BL, _f32(0.0), y)


def _lo2_call(lstm_out, fnh, pwl):
    return pl.pallas_call(
        _lo2_body,
        grid=(_LO_STEPS,),
        in_specs=[
            pl.BlockSpec((_LO_TREES, T, 128), lambda i: (jnp.minimum(i, B // _LO_TREES - 1), 0, 0)),
            pl.BlockSpec((_LO_TREES, HID), lambda i: (jnp.minimum(i, B // _LO_TREES - 1), 0)),
            pl.BlockSpec((128, HID), lambda i: (0, 0)),
        ],
        out_specs=pl.BlockSpec((_LO_BL, HID), lambda i: (i, 0)),
        out_shape=_sds((LO_PAD, HID)),
    )(lstm_out, fnh, pwl)


_SC_MESH = plsc.VectorSubcoreMesh(core_axis_name="c", subcore_axis_name="s")


@functools.partial(
    pl.kernel,
    out_type=[_sds((N, 16)), _sds((N, 16)), _sds((N, 16)), _sds((N, 16)),
              _sds((NPASS * N,), _i32)],
    mesh=_SC_MESH,
    compiler_params=pltpu.CompilerParams(needs_layout_passes=False, use_tc_tiling_on_sc=False),
    scratch_types=[
        pltpu.VMEM((TPAD, HID), _f32),
        pltpu.VMEM((TPAD, HID), _f32),
        pltpu.VMEM((TPAD, HID), _f32),
        pltpu.VMEM((TPAD, HID), _f32),
        pltpu.VMEM((GK,), _i32),
        pltpu.VMEM((GK,), _i32),
        pltpu.VMEM((GK,), _i32),
        pltpu.VMEM((GK,), _i32),
        pltpu.VMEM((GK,), _i32),
        pltpu.VMEM((GK,), _i32),
        pltpu.VMEM((GK // 128, 128), _i32),
        pltpu.VMEM((GK // 128, 128), _i32),
        pltpu.VMEM((GK // 128, 128), _i32),
        pltpu.VMEM((GK, HID), _f32),
        pltpu.VMEM((GK, HID), _f32),
        pltpu.VMEM((GK,), _i32),
        pltpu.VMEM((GK,), _i32),
        pltpu.VMEM((GK,), _i32),
        pltpu.SemaphoreType.DMA,
    ],
)
def _gather_kernel(nt_hbm, lt_hbm, pidx2_hbm, tid_hbm, pp_hbm, il_hbm, ip_hbm,
                   nt1_hbm, tn_hbm, tl1_hbm, tl2_hbm, tp_hbm, lo2_hbm,
                   base_lo_hbm, base_hi_hbm, pre_lo_hbm, pre_hi_hbm,
                   eff_all_hbm,
                   tn_v, tl1_v, tl2_v, tp_v,
                   nt_v, lt_v, tid_v, pp_v, il_v, ip_v,
                   pidx_v, ntp_v, flat_v, lo_v, pre_v,
                   e0_v, e1_v, e2_v, sem):
    wid = lax.axis_index("s") * NC + lax.axis_index("c")
    pltpu.sync_copy(tn_hbm, tn_v)
    pltpu.sync_copy(tl1_hbm, tl1_v)
    pltpu.sync_copy(tl2_hbm, tl2_v)
    pltpu.sync_copy(tp_hbm, tp_v)
    iota16 = lax.iota(_i32, 16)
    for chunk in range(NPW // GK):
        node0 = pl.multiple_of(wid * NPW + chunk * GK, GK)
        cps = [
            pltpu.async_copy(nt_hbm.at[pl.ds(node0, GK)], nt_v, sem),
            pltpu.async_copy(lt_hbm.at[pl.ds(node0, GK)], lt_v, sem),
            pltpu.async_copy(tid_hbm.at[pl.ds(node0, GK)], tid_v, sem),
            pltpu.async_copy(pp_hbm.at[pl.ds(node0, GK)], pp_v, sem),
            pltpu.async_copy(il_hbm.at[pl.ds(node0, GK)], il_v, sem),
            pltpu.async_copy(ip_hbm.at[pl.ds(node0, GK)], ip_v, sem),
            pltpu.async_copy(pidx2_hbm.at[pl.ds(pl.multiple_of(node0 // 128, 8), GK // 128)], pidx_v, sem),
        ]
        for cp in cps:
            cp.wait()

        def pass_a(g, carry):
            r = g // 8
            s0 = (g % 8) * 16
            tid = tid_v[pl.ds(g * 16, 16)]
            pp = pp_v[pl.ds(g * 16, 16)]
            il = il_v[pl.ds(g * 16, 16)]
            ip = ip_v[pl.ds(g * 16, 16)]
            mpb = il * ip
            dump = LO_ROWS + ((g * 16) % LO_SPREAD) + iota16
            flat = jnp.where(mpb == 1, tid * T + pp, dump)
            flat_v[r, pl.ds(s0, 16)] = flat
            x = pidx_v[r, pl.ds(s0, 16)]
            tr = PASS + jnp.bitwise_and(x, TRASH - 1)
            for pno, ev in ((0, e0_v), (1, e1_v), (2, e2_v)):
                lo_b = pno * PASS
                vhi = min(PASS, N - lo_b)
                inb = jnp.logical_and(x >= lo_b, x < lo_b + vhi)
                ev[pl.ds(g * 16, 16)] = jnp.where(inb, x - lo_b, tr)
            return carry

        lax.fori_loop(0, GK // 16, pass_a, 0)

        gcps = []
        for j in range(GK // 128):
            gcps.append(pltpu.async_copy(nt1_hbm.at[pidx_v.at[j]],
                                         ntp_v.at[j], sem))
            gcps.append(pltpu.async_copy(lo2_hbm.at[flat_v.at[j]],
                                         lo_v.at[pl.ds(j * 128, 128)], sem))
        for cp in gcps:
            cp.wait()

        def pass_b(g, carry):
            r = g // 8
            s0 = (g % 8) * 16
            nt = nt_v[pl.ds(g * 16, 16)]
            lt = lt_v[pl.ds(g * 16, 16)]
            il = il_v[pl.ds(g * 16, 16)]
            ip = ip_v[pl.ds(g * 16, 16)]
            ntp = ntp_v[r, pl.ds(s0, 16)]
            mlb = il * (1 - ip)
            eff_l1 = jnp.where(mlb == 1, nt, DICT)
            eff_l2 = jnp.where(mlb == 1, lt, DICT)
            eff_p = jnp.where(il == 1, DICT + 1, ntp)
            rows = g * 16 + iota16
            for c in range(HID):
                cvec = jnp.full((16,), c, _i32)
                v_l = (plsc.load_gather(tl1_v, [eff_l1, cvec]) +
                       plsc.load_gather(tl2_v, [eff_l2, cvec]))
                plsc.addupdate_scatter(lo_v, [rows, cvec], v_l)
                v_p = (plsc.load_gather(tn_v, [nt, cvec]) +
                       plsc.load_gather(tp_v, [eff_p, cvec]))
                plsc.store_scatter(pre_v, [rows, cvec], v_p)
            return carry

        lax.fori_loop(0, GK // 16, pass_b, 0)

        pltpu.sync_copy(lo_v.at[:, pl.ds(0, 16)], base_lo_hbm.at[pl.ds(node0, GK)])
        pltpu.sync_copy(lo_v.at[:, pl.ds(16, 16)], base_hi_hbm.at[pl.ds(node0, GK)])
        pltpu.sync_copy(pre_v.at[:, pl.ds(0, 16)], pre_lo_hbm.at[pl.ds(node0, GK)])
        pltpu.sync_copy(pre_v.at[:, pl.ds(16, 16)], pre_hi_hbm.at[pl.ds(node0, GK)])
        pltpu.sync_copy(e0_v, eff_all_hbm.at[pl.ds(node0, GK)])
        pltpu.sync_copy(e1_v, eff_all_hbm.at[pl.ds(N + node0, GK)])
        pltpu.sync_copy(e2_v, eff_all_hbm.at[pl.ds(2 * N + node0, GK)])


@functools.partial(
    pl.kernel,
    out_type=[_sds((N, 16)), _sds((N, 16))],
    mesh=_SC_MESH,
    compiler_params=pltpu.CompilerParams(needs_layout_passes=False, use_tc_tiling_on_sc=False),
    scratch_types=[
        pltpu.VMEM((SK, 16), _f32),
        pltpu.VMEM((SK // 128, 128), _i32),
        pltpu.VMEM((SK, 16), _f32),
        pltpu.VMEM_SHARED((PASS + TRASH, 16), _f32),
        pltpu.SemaphoreType.DMA,
    ],
)
def _scatter_kernel(h_lo_hbm, h_hi_hbm, eff_all_hbm,
                    cs_lo_hbm, cs_hi_hbm,
                    upd_v, eff_v, zb_v, acc, sem):
    cid = lax.axis_index("c")
    sid = lax.axis_index("s")
    z16 = jnp.zeros((16,), _f32)

    def zero_body(i, carry):
        zb_v[i, :] = z16
        return carry

    lax.fori_loop(0, SK, zero_body, 0)
    nchunks = N // NS // SK

    def run_core(h_hbm, out_hbm):
        for p in range(NPASS):
            lo_bound = p * PASS
            valid = min(PASS, N - p * PASS)
            rpt = (PASS + TRASH) // NS
            r0 = pl.multiple_of(sid * rpt, 8)
            pltpu.sync_copy(zb_v, acc.at[pl.ds(r0, SK)])
            pltpu.sync_copy(zb_v.at[pl.ds(0, rpt - SK)],
                            acc.at[pl.ds(r0 + SK, rpt - SK)])
            plsc.subcore_barrier()

            for chunk in range(nchunks):
                src0 = pl.multiple_of(sid * (N // NS) + chunk * SK, SK)
                cp1 = pltpu.async_copy(h_hbm.at[pl.ds(src0, SK)], upd_v, sem)
                cp2 = pltpu.async_copy(
                    eff_all_hbm.at[pl.ds(pl.multiple_of(
                        p * (N // 128) + src0 // 128, 8),
                        SK // 128)], eff_v, sem)
                cp1.wait()
                cp2.wait()

                def scat_body(j, carry):
                    j128 = pl.multiple_of(j * 128, 128)
                    pltpu.sync_copy(upd_v.at[pl.ds(j128, 128)],
                                    acc.at[eff_v.at[j]], add=True)
                    return carry

                lax.fori_loop(0, SK // 128, scat_body, 0)
            plsc.subcore_barrier()
            wb = valid // NS
            wb0 = pl.multiple_of(sid * wb, 8)
            pltpu.sync_copy(acc.at[pl.ds(wb0, wb)],
                            out_hbm.at[pl.ds(lo_bound + wb0, wb)])
            plsc.subcore_barrier()

    @pl.when(cid == 0)
    def _():
        run_core(h_lo_hbm, cs_lo_hbm)

    @pl.when(cid == 1)
    def _():
        run_core(h_hi_hbm, cs_hi_hbm)


@functools.partial(
    pl.kernel,
    out_type=[_sds((B, 16)), _sds((B, 16))],
    mesh=_SC_MESH,
    compiler_params=pltpu.CompilerParams(needs_layout_passes=False, use_tc_tiling_on_sc=False),
    scratch_types=[
        pltpu.VMEM((B // 128, 128), _i32),
        pltpu.VMEM((B, 16), _f32),
        pltpu.VMEM((B, 16), _f32),
        pltpu.SemaphoreType.DMA,
    ],
)
def _root_kernel(ridx2_hbm, h_lo_hbm, h_hi_hbm, g_lo_hbm, g_hi_hbm,
                 idx_v, lo_v, hi_v, sem):
    wid = lax.axis_index("s") * NC + lax.axis_index("c")

    @pl.when(wid == 0)
    def _():
        pltpu.sync_copy(ridx2_hbm, idx_v)
        for j in range(B // 128):
            pltpu.async_copy(h_lo_hbm.at[idx_v.at[j]],
                             lo_v.at[pl.ds(j * 128, 128)], sem).wait()
            pltpu.async_copy(h_hi_hbm.at[idx_v.at[j]],
                             hi_v.at[pl.ds(j * 128, 128)], sem).wait()
        pltpu.sync_copy(lo_v, g_lo_hbm)
        pltpu.sync_copy(hi_v, g_hi_hbm)


_UP_ROWS = N // 8
_UP_BL = 2048


def _update_body(cs_lo_ref, cs_hi_ref, base_lo_ref, base_hi_ref,
                 pre_lo_ref, pre_hi_ref, w11_ref, w21_ref, w12_ref, w22_ref,
                 h_lo_ref, h_hi_ref):
    cl = cs_lo_ref[...]
    ch = cs_hi_ref[...]
    t_lo = jnp.maximum(
        pre_lo_ref[...] +
        jnp.dot(cl, w11_ref[...], preferred_element_type=_f32) +
        jnp.dot(ch, w21_ref[...], preferred_element_type=_f32), _f32(0.0))
    t_hi = jnp.maximum(
        pre_hi_ref[...] +
        jnp.dot(cl, w12_ref[...], preferred_element_type=_f32) +
        jnp.dot(ch, w22_ref[...], preferred_element_type=_f32), _f32(0.0))
    h_lo_ref[...] = base_lo_ref[...] + t_lo
    h_hi_ref[...] = base_hi_ref[...] + t_hi


def _update_call(cs_lo, cs_hi, base_lo8, base_hi8, pre_lo8, pre_hi8,
                 w11, w21, w12, w22):
    bspec = pl.BlockSpec((_UP_BL, 128), lambda i: (i, 0))
    wspec = pl.BlockSpec((128, 128), lambda i: (0, 0))
    h_lo8, h_hi8 = pl.pallas_call(
        _update_body,
        grid=(_UP_ROWS // _UP_BL,),
        in_specs=[bspec, bspec, bspec, bspec, bspec, bspec,
                  wspec, wspec, wspec, wspec],
        out_specs=[bspec, bspec],
        out_shape=[_sds((_UP_ROWS, 128)), _sds((_UP_ROWS, 128))],
    )(cs_lo.reshape(_UP_ROWS, 128), cs_hi.reshape(_UP_ROWS, 128),
      base_lo8, base_hi8, pre_lo8, pre_hi8, w11, w21, w12, w22)
    return h_lo8.reshape(N, 16), h_hi8.reshape(N, 16)


def _head_body(g_lo_ref, g_hi_ref, w1_ref, b1_ref, w2_ref, b2_ref,
               tw_ref, tb_ref, out_ref):
    g = jnp.concatenate([g_lo_ref[...], g_hi_ref[...]], axis=1)
    t = jnp.dot(g, w1_ref[...], preferred_element_type=_f32) + b1_ref[...]
    t = jnp.dot(t, w2_ref[...], preferred_element_type=_f32) + b2_ref[...]
    out_ref[...] = jnp.dot(t, tw_ref[...], preferred_element_type=_f32) + tb_ref[...]


def _head_call(g_lo, g_hi, w1, b1, w2, b2, tw, tb):
    return pl.pallas_call(
        _head_body,
        out_shape=_sds((B, 1)),
    )(g_lo, g_hi, w1, b1, w2, b2, tw, tb)


def kernel(lstm_out, first_notes, node_type, leaf_token, parent_idx, is_leaf,
           is_ptr, tree_id, ptr_pos, root_idx, embedding, leaf_W1, leaf_b1,
           leaf_W2, leaf_b2, node_W, node_b, ptr_W, ptr_b, ff_W1, ff_b1,
           ff_W2, ff_b2, tail_W, tail_b):
    nt = node_type.astype(_i32)
    lt = leaf_token.astype(_i32)
    pidx = parent_idx.astype(_i32)
    tid = tree_id.astype(_i32)
    pp = ptr_pos.astype(_i32)
    il = is_leaf.astype(_i32)
    ip = is_ptr.astype(_i32)

    t_n, t_l1, t_l2, t_p, fnh, w11, w21, w12, w22 = _fold_call(
        embedding, leaf_W1[:ED], leaf_W1[ED:], leaf_W2,
        leaf_b1.reshape(1, HID), leaf_b2.reshape(1, HID),
        node_W[:ED], node_W[ED:2 * ED], node_b.reshape(1, HID),
        first_notes, ptr_W[:64], ptr_b.reshape(1, HID), node_W[2 * ED:])
    lo2 = _lo2_call(lstm_out, fnh, ptr_W[64:])

    pidx2 = pidx.reshape(N // 128, 128)
    base_lo, base_hi, pre_lo, pre_hi, eff_all = _gather_kernel(
        nt, lt, pidx2, tid, pp, il, ip, nt, t_n, t_l1, t_l2, t_p, lo2)
    eff_all = eff_all.reshape(NPASS * N // 128, 128)

    base_lo8 = base_lo.reshape(_UP_ROWS, 128)
    base_hi8 = base_hi.reshape(_UP_ROWS, 128)
    pre_lo8 = pre_lo.reshape(_UP_ROWS, 128)
    pre_hi8 = pre_hi.reshape(_UP_ROWS, 128)
    h_lo, h_hi = base_lo, base_hi
    for _ in range(8):
        cs_lo, cs_hi = _scatter_kernel(h_lo, h_hi, eff_all)
        h_lo, h_hi = _update_call(cs_lo, cs_hi, base_lo8, base_hi8,
                                  pre_lo8, pre_hi8, w11, w21, w12, w22)

    g_lo, g_hi = _root_kernel(root_idx.astype(_i32).reshape(B // 128, 128),
                              h_lo, h_hi)
    return _head_call(g_lo, g_hi, ff_W1, ff_b1.reshape(1, HID),
                      ff_W2, ff_b2.reshape(1, HID), tail_W,
                      tail_b.reshape(1, 1))


# speedup vs baseline: 6.5486x; 1.3327x over previous
import functools

import jax
import jax.numpy as jnp
from jax import lax
from jax.experimental import pallas as pl
from jax.experimental.pallas import tpu as pltpu
from jax.experimental.pallas import tpu_sc as plsc

N = 131072
B = 256
T = 256
HID = 32
ED = 16
DICT = 200
TPAD = 208
LO_ROWS = B * T
LO_SPREAD = 2048
LO_PAD = LO_ROWS + LO_SPREAD
BIG = 1e30

NC = 2
NS = 16
NW = NC * NS
NPW = N // NW
GK = 1024
SK = 2048
ZB = 2048
PASS = 44032
NPASS = 3
TRASH = 512

_f32 = jnp.float32
_i32 = jnp.int32


def _sds(shape, dtype=_f32):
    return jax.ShapeDtypeStruct(shape, dtype)


def _fold_body(emb_ref, lw1a_ref, lw1b_ref, lw2_ref, lb1_ref, lb2_ref,
               nwp_ref, nwn_ref, nb_ref, fn_ref, pwf_ref, pb_ref, wc_ref,
               tn_ref, tl1_ref, tl2_ref, tp_ref, fnh_ref,
               w11_ref, w21_ref, w12_ref, w22_ref):
    emb = emb_ref[...]
    sel = (lax.broadcasted_iota(_i32, (128, 16), 0) % 16 ==
           lax.broadcasted_iota(_i32, (128, 16), 1)).astype(_f32)
    selt = (lax.broadcasted_iota(_i32, (16, 128), 0) ==
            lax.broadcasted_iota(_i32, (16, 128), 1) % 16).astype(_f32)
    blk = (lax.broadcasted_iota(_i32, (128, 128), 0) // 16 ==
           lax.broadcasted_iota(_i32, (128, 128), 1) // 16).astype(_f32)
    wc = wc_ref[...]

    def packw(w16):
        t = jnp.dot(jnp.dot(sel, w16, preferred_element_type=_f32), selt,
                    preferred_element_type=_f32)
        return t * blk

    w11_ref[...] = packw(wc[:16, :16])
    w21_ref[...] = packw(wc[16:, :16])
    w12_ref[...] = packw(wc[:16, 16:])
    w22_ref[...] = packw(wc[16:, 16:])
    zpad = jnp.zeros((TPAD - DICT, HID), _f32)
    tn = jnp.dot(emb, nwn_ref[...], preferred_element_type=_f32)
    tn_ref[...] = jnp.concatenate([tn, zpad], axis=0)
    a1 = jnp.dot(lw1a_ref[...], lw2_ref[...], preferred_element_type=_f32)
    tl1 = jnp.dot(emb, a1, preferred_element_type=_f32)
    tl1_ref[...] = jnp.concatenate([tl1, zpad], axis=0)
    a2 = jnp.dot(lw1b_ref[...], lw2_ref[...], preferred_element_type=_f32)
    cl = jnp.dot(lb1_ref[...], lw2_ref[...], preferred_element_type=_f32) + lb2_ref[...]
    tl2 = jnp.dot(emb, a2, preferred_element_type=_f32) + cl
    tl2_ref[...] = jnp.concatenate([tl2, zpad], axis=0)
    tp = jnp.dot(emb, nwp_ref[...], preferred_element_type=_f32) + nb_ref[...]
    ridx = lax.broadcasted_iota(_i32, (TPAD - DICT, HID), 0)
    neg = jnp.where(ridx == 1, _f32(-BIG), _f32(0.0))
    tp_ref[...] = jnp.concatenate([tp, neg], axis=0)
    fnh_ref[...] = jnp.dot(fn_ref[...], pwf_ref[...], preferred_element_type=_f32) + pb_ref[...]


def _fold_call(emb, lw1a, lw1b, lw2, lb1, lb2, nwp, nwn, nb, fn, pwf, pb, wc):
    return pl.pallas_call(
        _fold_body,
        out_shape=[_sds((TPAD, HID)), _sds((TPAD, HID)), _sds((TPAD, HID)),
                   _sds((TPAD, HID)), _sds((B, HID)),
                   _sds((128, 128)), _sds((128, 128)), _sds((128, 128)),
                   _sds((128, 128))],
    )(emb, lw1a, lw1b, lw2, lb1, lb2, nwp, nwn, nb, fn, pwf, pb, wc)


_LO_TREES = 8
_LO_BL = _LO_TREES * T
_LO_STEPS = LO_PAD // _LO_BL


def _lo2_body(lstm_ref, fnh_ref, pwl_ref, out_ref):
    i = pl.program_id(0)
    x = lstm_ref[...].reshape(_LO_BL, 128)
    y = jnp.dot(x, pwl_ref[...], preferred_element_type=_f32)
    y = (y.reshape(_LO_TREES, T, HID) + fnh_ref[...][:, None, :]).reshape(_LO_BL, HID)
    out_ref[...] = jnp.where(i >= LO_ROWS // _LO_BL, _f32(0.0), y)


def _lo2_call(lstm_out, fnh, pwl):
    return pl.pallas_call(
        _lo2_body,
        grid=(_LO_STEPS,),
        in_specs=[
            pl.BlockSpec((_LO_TREES, T, 128), lambda i: (jnp.minimum(i, B // _LO_TREES - 1), 0, 0)),
            pl.BlockSpec((_LO_TREES, HID), lambda i: (jnp.minimum(i, B // _LO_TREES - 1), 0)),
            pl.BlockSpec((128, HID), lambda i: (0, 0)),
        ],
        out_specs=pl.BlockSpec((_LO_BL, HID), lambda i: (i, 0)),
        out_shape=_sds((LO_PAD, HID)),
    )(lstm_out, fnh, pwl)


_SC_MESH = plsc.VectorSubcoreMesh(core_axis_name="c", subcore_axis_name="s")


@functools.partial(
    pl.kernel,
    out_type=[_sds((N, 16)), _sds((N, 16)), _sds((N, 16)), _sds((N, 16)),
              _sds((NPASS * N,), _i32)],
    mesh=_SC_MESH,
    compiler_params=pltpu.CompilerParams(needs_layout_passes=False, use_tc_tiling_on_sc=False),
    scratch_types=[
        pltpu.VMEM((TPAD, HID), _f32),
        pltpu.VMEM((TPAD, HID), _f32),
        pltpu.VMEM((TPAD, HID), _f32),
        pltpu.VMEM((TPAD, HID), _f32),
        pltpu.VMEM((GK,), _i32),
        pltpu.VMEM((GK,), _i32),
        pltpu.VMEM((GK,), _i32),
        pltpu.VMEM((GK,), _i32),
        pltpu.VMEM((GK,), _i32),
        pltpu.VMEM((GK,), _i32),
        pltpu.VMEM((GK // 128, 128), _i32),
        pltpu.VMEM((GK // 128, 128), _i32),
        pltpu.VMEM((GK // 128, 128), _i32),
        pltpu.VMEM((GK, HID), _f32),
        pltpu.VMEM((GK, HID), _f32),
        pltpu.VMEM((GK,), _i32),
        pltpu.VMEM((GK,), _i32),
        pltpu.VMEM((GK,), _i32),
        pltpu.SemaphoreType.DMA,
    ],
)
def _gather_kernel(nt_hbm, lt_hbm, pidx2_hbm, tid_hbm, pp_hbm, il_hbm, ip_hbm,
                   nt1_hbm, tn_hbm, tl1_hbm, tl2_hbm, tp_hbm, lo2_hbm,
                   base_lo_hbm, base_hi_hbm, pre_lo_hbm, pre_hi_hbm,
                   eff_all_hbm,
                   tn_v, tl1_v, tl2_v, tp_v,
                   nt_v, lt_v, tid_v, pp_v, il_v, ip_v,
                   pidx_v, ntp_v, flat_v, lo_v, pre_v,
                   e0_v, e1_v, e2_v, sem):
    wid = lax.axis_index("s") * NC + lax.axis_index("c")
    pltpu.sync_copy(tn_hbm, tn_v)
    pltpu.sync_copy(tl1_hbm, tl1_v)
    pltpu.sync_copy(tl2_hbm, tl2_v)
    pltpu.sync_copy(tp_hbm, tp_v)
    iota16 = lax.iota(_i32, 16)
    for chunk in range(NPW // GK):
        node0 = pl.multiple_of(wid * NPW + chunk * GK, GK)
        cps = [
            pltpu.async_copy(nt_hbm.at[pl.ds(node0, GK)], nt_v, sem),
            pltpu.async_copy(lt_hbm.at[pl.ds(node0, GK)], lt_v, sem),
            pltpu.async_copy(tid_hbm.at[pl.ds(node0, GK)], tid_v, sem),
            pltpu.async_copy(pp_hbm.at[pl.ds(node0, GK)], pp_v, sem),
            pltpu.async_copy(il_hbm.at[pl.ds(node0, GK)], il_v, sem),
            pltpu.async_copy(ip_hbm.at[pl.ds(node0, GK)], ip_v, sem),
            pltpu.async_copy(pidx2_hbm.at[pl.ds(pl.multiple_of(node0 // 128, 8), GK // 128)], pidx_v, sem),
        ]
        for cp in cps:
            cp.wait()

        def pass_a(g, carry):
            r = g // 8
            s0 = (g % 8) * 16
            tid = tid_v[pl.ds(g * 16, 16)]
            pp = pp_v[pl.ds(g * 16, 16)]
            il = il_v[pl.ds(g * 16, 16)]
            ip = ip_v[pl.ds(g * 16, 16)]
            mpb = il * ip
            dump = LO_ROWS + ((g * 16) % LO_SPREAD) + iota16
            flat = jnp.where(mpb == 1, tid * T + pp, dump)
            flat_v[r, pl.ds(s0, 16)] = flat
            x = pidx_v[r, pl.ds(s0, 16)]
            tr = PASS + jnp.bitwise_and(x, TRASH - 1)
            for pno, ev in ((0, e0_v), (1, e1_v), (2, e2_v)):
                lo_b = pno * PASS
                vhi = min(PASS, N - lo_b)
                inb = jnp.logical_and(x >= lo_b, x < lo_b + vhi)
                ev[pl.ds(g * 16, 16)] = jnp.where(inb, x - lo_b, tr)
            return carry

        lax.fori_loop(0, GK // 16, pass_a, 0)

        gcps = []
        for j in range(GK // 128):
            gcps.append(pltpu.async_copy(nt1_hbm.at[pidx_v.at[j]],
                                         ntp_v.at[j], sem))
            gcps.append(pltpu.async_copy(lo2_hbm.at[flat_v.at[j]],
                                         lo_v.at[pl.ds(j * 128, 128)], sem))
        for cp in gcps:
            cp.wait()

        def pass_b(g, carry):
            r = g // 8
            s0 = (g % 8) * 16
            nt = nt_v[pl.ds(g * 16, 16)]
            lt = lt_v[pl.ds(g * 16, 16)]
            il = il_v[pl.ds(g * 16, 16)]
            ip = ip_v[pl.ds(g * 16, 16)]
            ntp = ntp_v[r, pl.ds(s0, 16)]
            mlb = il * (1 - ip)
            eff_l1 = jnp.where(mlb == 1, nt, DICT)
            eff_l2 = jnp.where(mlb == 1, lt, DICT)
            eff_p = jnp.where(il == 1, DICT + 1, ntp)
            for lane in range(16):
                node = g * 16 + lane
                l1 = eff_l1[lane]
                l2 = eff_l2[lane]
                pn = nt[lane]
                pr = eff_p[lane]
                for h in (0, 16):
                    b = tl1_v[l1, pl.ds(h, 16)] + tl2_v[l2, pl.ds(h, 16)]
                    lo_v[node, pl.ds(h, 16)] = lo_v[node, pl.ds(h, 16)] + b
                    pre_v[node, pl.ds(h, 16)] = (tn_v[pn, pl.ds(h, 16)] +
                                                 tp_v[pr, pl.ds(h, 16)])
            return carry

        lax.fori_loop(0, GK // 16, pass_b, 0)

        pltpu.sync_copy(lo_v.at[:, pl.ds(0, 16)], base_lo_hbm.at[pl.ds(node0, GK)])
        pltpu.sync_copy(lo_v.at[:, pl.ds(16, 16)], base_hi_hbm.at[pl.ds(node0, GK)])
        pltpu.sync_copy(pre_v.at[:, pl.ds(0, 16)], pre_lo_hbm.at[pl.ds(node0, GK)])
        pltpu.sync_copy(pre_v.at[:, pl.ds(16, 16)], pre_hi_hbm.at[pl.ds(node0, GK)])
        pltpu.sync_copy(e0_v, eff_all_hbm.at[pl.ds(node0, GK)])
        pltpu.sync_copy(e1_v, eff_all_hbm.at[pl.ds(N + node0, GK)])
        pltpu.sync_copy(e2_v, eff_all_hbm.at[pl.ds(2 * N + node0, GK)])


@functools.partial(
    pl.kernel,
    out_type=[_sds((N, 16)), _sds((N, 16))],
    mesh=_SC_MESH,
    compiler_params=pltpu.CompilerParams(needs_layout_passes=False, use_tc_tiling_on_sc=False),
    scratch_types=[
        pltpu.VMEM((SK, 16), _f32),
        pltpu.VMEM((SK // 128, 128), _i32),
        pltpu.VMEM((ZB, 16), _f32),
        pltpu.VMEM_SHARED((PASS + TRASH, 16), _f32),
        pltpu.SemaphoreType.DMA,
    ],
)
def _scatter_kernel(h_lo_hbm, h_hi_hbm, eff_all_hbm,
                    cs_lo_hbm, cs_hi_hbm,
                    upd_v, eff_v, zb_v, acc, sem):
    cid = lax.axis_index("c")
    sid = lax.axis_index("s")
    z16 = jnp.zeros((16,), _f32)

    def zero_body(i, carry):
        zb_v[i, :] = z16
        return carry

    lax.fori_loop(0, ZB, zero_body, 0)
    nchunks = N // NS // SK

    def run_core(h_hbm, out_hbm):
        for p in range(NPASS):
            lo_bound = p * PASS
            valid = min(PASS, N - p * PASS)
            rpt = (PASS + TRASH) // NS
            r0 = pl.multiple_of(sid * rpt, 8)
            pltpu.sync_copy(zb_v, acc.at[pl.ds(r0, ZB)])
            pltpu.sync_copy(zb_v.at[pl.ds(0, rpt - ZB)],
                            acc.at[pl.ds(r0 + ZB, rpt - ZB)])
            plsc.subcore_barrier()

            for chunk in range(nchunks):
                src0 = pl.multiple_of(sid * (N // NS) + chunk * SK, SK)
                cp1 = pltpu.async_copy(h_hbm.at[pl.ds(src0, SK)], upd_v, sem)
                cp2 = pltpu.async_copy(
                    eff_all_hbm.at[pl.ds(pl.multiple_of(
                        p * (N // 128) + src0 // 128, 8),
                        SK // 128)], eff_v, sem)
                cp1.wait()
                cp2.wait()

                def scat_body(j, carry):
                    j128 = pl.multiple_of(j * 128, 128)
                    pltpu.sync_copy(upd_v.at[pl.ds(j128, 128)],
                                    acc.at[eff_v.at[j]], add=True)
                    return carry

                lax.fori_loop(0, SK // 128, scat_body, 0)
            plsc.subcore_barrier()
            wb = valid // NS
            wb0 = pl.multiple_of(sid * wb, 8)
            pltpu.sync_copy(acc.at[pl.ds(wb0, wb)],
                            out_hbm.at[pl.ds(lo_bound + wb0, wb)])
            plsc.subcore_barrier()

    @pl.when(cid == 0)
    def _():
        run_core(h_lo_hbm, cs_lo_hbm)

    @pl.when(cid == 1)
    def _():
        run_core(h_hi_hbm, cs_hi_hbm)


@functools.partial(
    pl.kernel,
    out_type=[_sds((B, 16)), _sds((B, 16))],
    mesh=_SC_MESH,
    compiler_params=pltpu.CompilerParams(needs_layout_passes=False, use_tc_tiling_on_sc=False),
    scratch_types=[
        pltpu.VMEM((B // 128, 128), _i32),
        pltpu.VMEM((B, 16), _f32),
        pltpu.VMEM((B, 16), _f32),
        pltpu.SemaphoreType.DMA,
    ],
)
def _root_kernel(ridx2_hbm, h_lo_hbm, h_hi_hbm, g_lo_hbm, g_hi_hbm,
                 idx_v, lo_v, hi_v, sem):
    wid = lax.axis_index("s") * NC + lax.axis_index("c")

    @pl.when(wid == 0)
    def _():
        pltpu.sync_copy(ridx2_hbm, idx_v)
        for j in range(B // 128):
            pltpu.async_copy(h_lo_hbm.at[idx_v.at[j]],
                             lo_v.at[pl.ds(j * 128, 128)], sem).wait()
            pltpu.async_copy(h_hi_hbm.at[idx_v.at[j]],
                             hi_v.at[pl.ds(j * 128, 128)], sem).wait()
        pltpu.sync_copy(lo_v, g_lo_hbm)
        pltpu.sync_copy(hi_v, g_hi_hbm)


_UP_ROWS = N // 8
_UP_BL = 2048


def _update_body(cs_lo_ref, cs_hi_ref, base_lo_ref, base_hi_ref,
                 pre_lo_ref, pre_hi_ref, w11_ref, w21_ref, w12_ref, w22_ref,
                 h_lo_ref, h_hi_ref):
    cl = cs_lo_ref[...]
    ch = cs_hi_ref[...]
    t_lo = jnp.maximum(
        pre_lo_ref[...] +
        jnp.dot(cl, w11_ref[...], preferred_element_type=_f32) +
        jnp.dot(ch, w21_ref[...], preferred_element_type=_f32), _f32(0.0))
    t_hi = jnp.maximum(
        pre_hi_ref[...] +
        jnp.dot(cl, w12_ref[...], preferred_element_type=_f32) +
        jnp.dot(ch, w22_ref[...], preferred_element_type=_f32), _f32(0.0))
    h_lo_ref[...] = base_lo_ref[...] + t_lo
    h_hi_ref[...] = base_hi_ref[...] + t_hi


def _update_call(cs_lo, cs_hi, base_lo8, base_hi8, pre_lo8, pre_hi8,
                 w11, w21, w12, w22):
    bspec = pl.BlockSpec((_UP_BL, 128), lambda i: (i, 0))
    wspec = pl.BlockSpec((128, 128), lambda i: (0, 0))
    h_lo8, h_hi8 = pl.pallas_call(
        _update_body,
        grid=(_UP_ROWS // _UP_BL,),
        in_specs=[bspec, bspec, bspec, bspec, bspec, bspec,
                  wspec, wspec, wspec, wspec],
        out_specs=[bspec, bspec],
        out_shape=[_sds((_UP_ROWS, 128)), _sds((_UP_ROWS, 128))],
    )(cs_lo.reshape(_UP_ROWS, 128), cs_hi.reshape(_UP_ROWS, 128),
      base_lo8, base_hi8, pre_lo8, pre_hi8, w11, w21, w12, w22)
    return h_lo8.reshape(N, 16), h_hi8.reshape(N, 16)


def _head_body(g_lo_ref, g_hi_ref, w1_ref, b1_ref, w2_ref, b2_ref,
               tw_ref, tb_ref, out_ref):
    g = jnp.concatenate([g_lo_ref[...], g_hi_ref[...]], axis=1)
    t = jnp.dot(g, w1_ref[...], preferred_element_type=_f32) + b1_ref[...]
    t = jnp.dot(t, w2_ref[...], preferred_element_type=_f32) + b2_ref[...]
    out_ref[...] = jnp.dot(t, tw_ref[...], preferred_element_type=_f32) + tb_ref[...]


def _head_call(g_lo, g_hi, w1, b1, w2, b2, tw, tb):
    return pl.pallas_call(
        _head_body,
        out_shape=_sds((B, 1)),
    )(g_lo, g_hi, w1, b1, w2, b2, tw, tb)


def kernel(lstm_out, first_notes, node_type, leaf_token, parent_idx, is_leaf,
           is_ptr, tree_id, ptr_pos, root_idx, embedding, leaf_W1, leaf_b1,
           leaf_W2, leaf_b2, node_W, node_b, ptr_W, ptr_b, ff_W1, ff_b1,
           ff_W2, ff_b2, tail_W, tail_b):
    nt = node_type.astype(_i32)
    lt = leaf_token.astype(_i32)
    pidx = parent_idx.astype(_i32)
    tid = tree_id.astype(_i32)
    pp = ptr_pos.astype(_i32)
    il = is_leaf.astype(_i32)
    ip = is_ptr.astype(_i32)

    t_n, t_l1, t_l2, t_p, fnh, w11, w21, w12, w22 = _fold_call(
        embedding, leaf_W1[:ED], leaf_W1[ED:], leaf_W2,
        leaf_b1.reshape(1, HID), leaf_b2.reshape(1, HID),
        node_W[:ED], node_W[ED:2 * ED], node_b.reshape(1, HID),
        first_notes, ptr_W[:64], ptr_b.reshape(1, HID), node_W[2 * ED:])
    lo2 = _lo2_call(lstm_out, fnh, ptr_W[64:])

    pidx2 = pidx.reshape(N // 128, 128)
    base_lo, base_hi, pre_lo, pre_hi, eff_all = _gather_kernel(
        nt, lt, pidx2, tid, pp, il, ip, nt, t_n, t_l1, t_l2, t_p, lo2)
    eff_all = eff_all.reshape(NPASS * N // 128, 128)

    base_lo8 = base_lo.reshape(_UP_ROWS, 128)
    base_hi8 = base_hi.reshape(_UP_ROWS, 128)
    pre_lo8 = pre_lo.reshape(_UP_ROWS, 128)
    pre_hi8 = pre_hi.reshape(_UP_ROWS, 128)
    h_lo, h_hi = base_lo, base_hi
    for _ in range(8):
        cs_lo, cs_hi = _scatter_kernel(h_lo, h_hi, eff_all)
        h_lo, h_hi = _update_call(cs_lo, cs_hi, base_lo8, base_hi8,
                                  pre_lo8, pre_hi8, w11, w21, w12, w22)

    g_lo, g_hi = _root_kernel(root_idx.astype(_i32).reshape(B // 128, 128),
                              h_lo, h_hi)
    return _head_call(g_lo, g_hi, ff_W1, ff_b1.reshape(1, HID),
                      ff_W2, ff_b2.reshape(1, HID), tail_W,
                      tail_b.reshape(1, 1))


# speedup vs baseline: 7.0939x; 1.0833x over previous
import functools

import jax
import jax.numpy as jnp
from jax import lax
from jax.experimental import pallas as pl
from jax.experimental.pallas import tpu as pltpu
from jax.experimental.pallas import tpu_sc as plsc

N = 131072
B = 256
T = 256
HID = 32
ED = 16
DICT = 200
TPAD = 208
LO_ROWS = B * T
LO_SPREAD = 2048
LO_PAD = LO_ROWS + LO_SPREAD
BIG = 1e30

NC = 2
NS = 16
NW = NC * NS
NPW = N // NW
GK = 1024
SK = 2048
ZB = 2048
PASS = 44032
NPASS = 3
TRASH = 512

_f32 = jnp.float32
_i32 = jnp.int32


def _sds(shape, dtype=_f32):
    return jax.ShapeDtypeStruct(shape, dtype)


def _fold_body(emb_ref, lw1a_ref, lw1b_ref, lw2_ref, lb1_ref, lb2_ref,
               nwp_ref, nwn_ref, nb_ref, fn_ref, pwf_ref, pb_ref, wc_ref,
               tn_ref, tl1_ref, tl2_ref, tp_ref, fnh_ref,
               w11_ref, w21_ref, w12_ref, w22_ref):
    emb = emb_ref[...]
    sel = (lax.broadcasted_iota(_i32, (128, 16), 0) % 16 ==
           lax.broadcasted_iota(_i32, (128, 16), 1)).astype(_f32)
    selt = (lax.broadcasted_iota(_i32, (16, 128), 0) ==
            lax.broadcasted_iota(_i32, (16, 128), 1) % 16).astype(_f32)
    blk = (lax.broadcasted_iota(_i32, (128, 128), 0) // 16 ==
           lax.broadcasted_iota(_i32, (128, 128), 1) // 16).astype(_f32)
    wc = wc_ref[...]

    def packw(w16):
        t = jnp.dot(jnp.dot(sel, w16, preferred_element_type=_f32), selt,
                    preferred_element_type=_f32)
        return t * blk

    w11_ref[...] = packw(wc[:16, :16])
    w21_ref[...] = packw(wc[16:, :16])
    w12_ref[...] = packw(wc[:16, 16:])
    w22_ref[...] = packw(wc[16:, 16:])
    zpad = jnp.zeros((TPAD - DICT, HID), _f32)
    tn = jnp.dot(emb, nwn_ref[...], preferred_element_type=_f32)
    tn_ref[...] = jnp.concatenate([tn, zpad], axis=0)
    a1 = jnp.dot(lw1a_ref[...], lw2_ref[...], preferred_element_type=_f32)
    tl1 = jnp.dot(emb, a1, preferred_element_type=_f32)
    tl1_ref[...] = jnp.concatenate([tl1, zpad], axis=0)
    a2 = jnp.dot(lw1b_ref[...], lw2_ref[...], preferred_element_type=_f32)
    cl = jnp.dot(lb1_ref[...], lw2_ref[...], preferred_element_type=_f32) + lb2_ref[...]
    tl2 = jnp.dot(emb, a2, preferred_element_type=_f32) + cl
    tl2_ref[...] = jnp.concatenate([tl2, zpad], axis=0)
    tp = jnp.dot(emb, nwp_ref[...], preferred_element_type=_f32) + nb_ref[...]
    ridx = lax.broadcasted_iota(_i32, (TPAD - DICT, HID), 0)
    neg = jnp.where(ridx == 1, _f32(-BIG), _f32(0.0))
    tp_ref[...] = jnp.concatenate([tp, neg], axis=0)
    fnh_ref[...] = jnp.dot(fn_ref[...], pwf_ref[...], preferred_element_type=_f32) + pb_ref[...]


def _fold_call(emb, lw1a, lw1b, lw2, lb1, lb2, nwp, nwn, nb, fn, pwf, pb, wc):
    return pl.pallas_call(
        _fold_body,
        out_shape=[_sds((TPAD, HID)), _sds((TPAD, HID)), _sds((TPAD, HID)),
                   _sds((TPAD, HID)), _sds((B, HID)),
                   _sds((128, 128)), _sds((128, 128)), _sds((128, 128)),
                   _sds((128, 128))],
    )(emb, lw1a, lw1b, lw2, lb1, lb2, nwp, nwn, nb, fn, pwf, pb, wc)


_LO_TREES = 8
_LO_BL = _LO_TREES * T
_LO_STEPS = LO_PAD // _LO_BL


def _lo2_body(lstm_ref, fnh_ref, pwl_ref, out_ref):
    i = pl.program_id(0)
    x = lstm_ref[...].reshape(_LO_BL, 128)
    y = jnp.dot(x, pwl_ref[...], preferred_element_type=_f32)
    y = (y.reshape(_LO_TREES, T, HID) + fnh_ref[...][:, None, :]).reshape(_LO_BL, HID)
    out_ref[...] = jnp.where(i >= LO_ROWS // _LO_BL, _f32(0.0), y)


def _lo2_call(lstm_out, fnh, pwl):
    return pl.pallas_call(
        _lo2_body,
        grid=(_LO_STEPS,),
        in_specs=[
            pl.BlockSpec((_LO_TREES, T, 128), lambda i: (jnp.minimum(i, B // _LO_TREES - 1), 0, 0)),
            pl.BlockSpec((_LO_TREES, HID), lambda i: (jnp.minimum(i, B // _LO_TREES - 1), 0)),
            pl.BlockSpec((128, HID), lambda i: (0, 0)),
        ],
        out_specs=pl.BlockSpec((_LO_BL, HID), lambda i: (i, 0)),
        out_shape=_sds((LO_PAD, HID)),
    )(lstm_out, fnh, pwl)


_SC_MESH = plsc.VectorSubcoreMesh(core_axis_name="c", subcore_axis_name="s")


@functools.partial(
    pl.kernel,
    out_type=[_sds((N, 16)), _sds((N, 16)), _sds((N, 16)), _sds((N, 16)),
              _sds((NPASS * N,), _i32)],
    mesh=_SC_MESH,
    compiler_params=pltpu.CompilerParams(needs_layout_passes=False, use_tc_tiling_on_sc=False),
    scratch_types=[
        pltpu.VMEM((TPAD, HID), _f32),
        pltpu.VMEM((TPAD, HID), _f32),
        pltpu.VMEM((TPAD, HID), _f32),
        pltpu.VMEM((TPAD, HID), _f32),
        pltpu.VMEM((GK,), _i32),
        pltpu.VMEM((GK,), _i32),
        pltpu.VMEM((GK,), _i32),
        pltpu.VMEM((GK,), _i32),
        pltpu.VMEM((GK,), _i32),
        pltpu.VMEM((GK,), _i32),
        pltpu.VMEM((GK // 128, 128), _i32),
        pltpu.VMEM((GK // 128, 128), _i32),
        pltpu.VMEM((GK // 128, 128), _i32),
        pltpu.VMEM((GK, HID), _f32),
        pltpu.VMEM((GK, HID), _f32),
        pltpu.VMEM((GK,), _i32),
        pltpu.VMEM((GK,), _i32),
        pltpu.VMEM((GK,), _i32),
        pltpu.SemaphoreType.DMA,
    ],
)
def _gather_kernel(nt_hbm, lt_hbm, pidx2_hbm, tid_hbm, pp_hbm, il_hbm, ip_hbm,
                   nt1_hbm, tn_hbm, tl1_hbm, tl2_hbm, tp_hbm, lo2_hbm,
                   base_lo_hbm, base_hi_hbm, pre_lo_hbm, pre_hi_hbm,
                   eff_all_hbm,
                   tn_v, tl1_v, tl2_v, tp_v,
                   nt_v, lt_v, tid_v, pp_v, il_v, ip_v,
                   pidx_v, ntp_v, flat_v, lo_v, pre_v,
                   e0_v, e1_v, e2_v, sem):
    wid = lax.axis_index("s") * NC + lax.axis_index("c")
    pltpu.sync_copy(tn_hbm, tn_v)
    pltpu.sync_copy(tl1_hbm, tl1_v)
    pltpu.sync_copy(tl2_hbm, tl2_v)
    pltpu.sync_copy(tp_hbm, tp_v)
    iota16 = lax.iota(_i32, 16)
    for chunk in range(NPW // GK):
        node0 = pl.multiple_of(wid * NPW + chunk * GK, GK)
        cps = [
            pltpu.async_copy(nt_hbm.at[pl.ds(node0, GK)], nt_v, sem),
            pltpu.async_copy(lt_hbm.at[pl.ds(node0, GK)], lt_v, sem),
            pltpu.async_copy(tid_hbm.at[pl.ds(node0, GK)], tid_v, sem),
            pltpu.async_copy(pp_hbm.at[pl.ds(node0, GK)], pp_v, sem),
            pltpu.async_copy(il_hbm.at[pl.ds(node0, GK)], il_v, sem),
            pltpu.async_copy(ip_hbm.at[pl.ds(node0, GK)], ip_v, sem),
            pltpu.async_copy(pidx2_hbm.at[pl.ds(pl.multiple_of(node0 // 128, 8), GK // 128)], pidx_v, sem),
        ]
        for cp in cps:
            cp.wait()

        def pass_a(g, carry):
            r = g // 8
            s0 = (g % 8) * 16
            tid = tid_v[pl.ds(g * 16, 16)]
            pp = pp_v[pl.ds(g * 16, 16)]
            il = il_v[pl.ds(g * 16, 16)]
            ip = ip_v[pl.ds(g * 16, 16)]
            mpb = il * ip
            dump = LO_ROWS + ((g * 16) % LO_SPREAD) + iota16
            flat = jnp.where(mpb == 1, tid * T + pp, dump)
            flat_v[r, pl.ds(s0, 16)] = flat
            x = pidx_v[r, pl.ds(s0, 16)]
            tr = PASS + jnp.bitwise_and(x, TRASH - 1)
            for pno, ev in ((0, e0_v), (1, e1_v), (2, e2_v)):
                lo_b = pno * PASS
                vhi = min(PASS, N - lo_b)
                inb = jnp.logical_and(x >= lo_b, x < lo_b + vhi)
                ev[pl.ds(g * 16, 16)] = jnp.where(inb, x - lo_b, tr)
            return carry

        lax.fori_loop(0, GK // 16, pass_a, 0)

        gcps = []
        for j in range(GK // 128):
            gcps.append(pltpu.async_copy(nt1_hbm.at[pidx_v.at[j]],
                                         ntp_v.at[j], sem))
            gcps.append(pltpu.async_copy(lo2_hbm.at[flat_v.at[j]],
                                         lo_v.at[pl.ds(j * 128, 128)], sem))
        for cp in gcps:
            cp.wait()

        def pass_b(g, carry):
            r = g // 8
            s0 = (g % 8) * 16
            nt = nt_v[pl.ds(g * 16, 16)]
            lt = lt_v[pl.ds(g * 16, 16)]
            il = il_v[pl.ds(g * 16, 16)]
            ip = ip_v[pl.ds(g * 16, 16)]
            ntp = ntp_v[r, pl.ds(s0, 16)]
            mlb = il * (1 - ip)
            eff_l1 = jnp.where(mlb == 1, nt, DICT)
            eff_l2 = jnp.where(mlb == 1, lt, DICT)
            eff_p = jnp.where(il == 1, DICT + 1, ntp)
            for lane in range(16):
                node = g * 16 + lane
                l1 = eff_l1[lane]
                l2 = eff_l2[lane]
                pn = nt[lane]
                pr = eff_p[lane]
                for h in (0, 16):
                    b = tl1_v[l1, pl.ds(h, 16)] + tl2_v[l2, pl.ds(h, 16)]
                    lo_v[node, pl.ds(h, 16)] = lo_v[node, pl.ds(h, 16)] + b
                    pre_v[node, pl.ds(h, 16)] = (tn_v[pn, pl.ds(h, 16)] +
                                                 tp_v[pr, pl.ds(h, 16)])
            return carry

        lax.fori_loop(0, GK // 16, pass_b, 0)

        pltpu.sync_copy(lo_v.at[:, pl.ds(0, 16)], base_lo_hbm.at[pl.ds(node0, GK)])
        pltpu.sync_copy(lo_v.at[:, pl.ds(16, 16)], base_hi_hbm.at[pl.ds(node0, GK)])
        pltpu.sync_copy(pre_v.at[:, pl.ds(0, 16)], pre_lo_hbm.at[pl.ds(node0, GK)])
        pltpu.sync_copy(pre_v.at[:, pl.ds(16, 16)], pre_hi_hbm.at[pl.ds(node0, GK)])
        pltpu.sync_copy(e0_v, eff_all_hbm.at[pl.ds(node0, GK)])
        pltpu.sync_copy(e1_v, eff_all_hbm.at[pl.ds(N + node0, GK)])
        pltpu.sync_copy(e2_v, eff_all_hbm.at[pl.ds(2 * N + node0, GK)])


@functools.partial(
    pl.kernel,
    out_type=[_sds((N, 16)), _sds((N, 16))],
    mesh=_SC_MESH,
    compiler_params=pltpu.CompilerParams(needs_layout_passes=False, use_tc_tiling_on_sc=False),
    scratch_types=[
        pltpu.VMEM((SK, 16), _f32),
        pltpu.VMEM((SK // 128, 128), _i32),
        pltpu.VMEM((ZB, 16), _f32),
        pltpu.VMEM_SHARED((PASS + TRASH, 16), _f32),
        pltpu.SemaphoreType.DMA,
    ],
)
def _scatter_kernel(h_lo_hbm, h_hi_hbm, eff_all_hbm,
                    cs_lo_hbm, cs_hi_hbm,
                    upd_v, eff_v, zb_v, acc, sem):
    cid = lax.axis_index("c")
    sid = lax.axis_index("s")
    z16 = jnp.zeros((16,), _f32)

    def zero_body(i, carry):
        zb_v[i, :] = z16
        return carry

    lax.fori_loop(0, ZB, zero_body, 0)
    nchunks = N // NS // SK

    def run_core(h_hbm, out_hbm):
        for p in range(NPASS):
            lo_bound = p * PASS
            valid = min(PASS, N - p * PASS)
            rpt = (PASS + TRASH) // NS
            r0 = pl.multiple_of(sid * rpt, 8)
            pltpu.sync_copy(zb_v, acc.at[pl.ds(r0, ZB)])
            pltpu.sync_copy(zb_v.at[pl.ds(0, rpt - ZB)],
                            acc.at[pl.ds(r0 + ZB, rpt - ZB)])
            plsc.subcore_barrier()

            for chunk in range(nchunks):
                src0 = pl.multiple_of(sid * (N // NS) + chunk * SK, SK)
                cp1 = pltpu.async_copy(h_hbm.at[pl.ds(src0, SK)], upd_v, sem)
                cp2 = pltpu.async_copy(
                    eff_all_hbm.at[pl.ds(pl.multiple_of(
                        p * (N // 128) + src0 // 128, 8),
                        SK // 128)], eff_v, sem)
                cp1.wait()
                cp2.wait()

                scs = [pltpu.async_copy(upd_v.at[pl.ds(j * 128, 128)],
                                        acc.at[eff_v.at[j]], sem, add=True)
                       for j in range(SK // 128)]
                for cp in scs:
                    cp.wait()
            plsc.subcore_barrier()
            wb = valid // NS
            wb0 = pl.multiple_of(sid * wb, 8)
            pltpu.sync_copy(acc.at[pl.ds(wb0, wb)],
                            out_hbm.at[pl.ds(lo_bound + wb0, wb)])
            plsc.subcore_barrier()

    @pl.when(cid == 0)
    def _():
        run_core(h_lo_hbm, cs_lo_hbm)

    @pl.when(cid == 1)
    def _():
        run_core(h_hi_hbm, cs_hi_hbm)


@functools.partial(
    pl.kernel,
    out_type=[_sds((B, 16)), _sds((B, 16))],
    mesh=_SC_MESH,
    compiler_params=pltpu.CompilerParams(needs_layout_passes=False, use_tc_tiling_on_sc=False),
    scratch_types=[
        pltpu.VMEM((B // 128, 128), _i32),
        pltpu.VMEM((B, 16), _f32),
        pltpu.VMEM((B, 16), _f32),
        pltpu.SemaphoreType.DMA,
    ],
)
def _root_kernel(ridx2_hbm, h_lo_hbm, h_hi_hbm, g_lo_hbm, g_hi_hbm,
                 idx_v, lo_v, hi_v, sem):
    wid = lax.axis_index("s") * NC + lax.axis_index("c")

    @pl.when(wid == 0)
    def _():
        pltpu.sync_copy(ridx2_hbm, idx_v)
        for j in range(B // 128):
            pltpu.async_copy(h_lo_hbm.at[idx_v.at[j]],
                             lo_v.at[pl.ds(j * 128, 128)], sem).wait()
            pltpu.async_copy(h_hi_hbm.at[idx_v.at[j]],
                             hi_v.at[pl.ds(j * 128, 128)], sem).wait()
        pltpu.sync_copy(lo_v, g_lo_hbm)
        pltpu.sync_copy(hi_v, g_hi_hbm)


_UP_ROWS = N // 8
_UP_BL = 2048


def _update_body(cs_lo_ref, cs_hi_ref, base_lo_ref, base_hi_ref,
                 pre_lo_ref, pre_hi_ref, w11_ref, w21_ref, w12_ref, w22_ref,
                 h_lo_ref, h_hi_ref):
    cl = cs_lo_ref[...]
    ch = cs_hi_ref[...]
    t_lo = jnp.maximum(
        pre_lo_ref[...] +
        jnp.dot(cl, w11_ref[...], preferred_element_type=_f32) +
        jnp.dot(ch, w21_ref[...], preferred_element_type=_f32), _f32(0.0))
    t_hi = jnp.maximum(
        pre_hi_ref[...] +
        jnp.dot(cl, w12_ref[...], preferred_element_type=_f32) +
        jnp.dot(ch, w22_ref[...], preferred_element_type=_f32), _f32(0.0))
    h_lo_ref[...] = base_lo_ref[...] + t_lo
    h_hi_ref[...] = base_hi_ref[...] + t_hi


def _update_call(cs_lo, cs_hi, base_lo8, base_hi8, pre_lo8, pre_hi8,
                 w11, w21, w12, w22):
    bspec = pl.BlockSpec((_UP_BL, 128), lambda i: (i, 0))
    wspec = pl.BlockSpec((128, 128), lambda i: (0, 0))
    h_lo8, h_hi8 = pl.pallas_call(
        _update_body,
        grid=(_UP_ROWS // _UP_BL,),
        in_specs=[bspec, bspec, bspec, bspec, bspec, bspec,
                  wspec, wspec, wspec, wspec],
        out_specs=[bspec, bspec],
        out_shape=[_sds((_UP_ROWS, 128)), _sds((_UP_ROWS, 128))],
    )(cs_lo.reshape(_UP_ROWS, 128), cs_hi.reshape(_UP_ROWS, 128),
      base_lo8, base_hi8, pre_lo8, pre_hi8, w11, w21, w12, w22)
    return h_lo8.reshape(N, 16), h_hi8.reshape(N, 16)


def _head_body(g_lo_ref, g_hi_ref, w1_ref, b1_ref, w2_ref, b2_ref,
               tw_ref, tb_ref, out_ref):
    g = jnp.concatenate([g_lo_ref[...], g_hi_ref[...]], axis=1)
    t = jnp.dot(g, w1_ref[...], preferred_element_type=_f32) + b1_ref[...]
    t = jnp.dot(t, w2_ref[...], preferred_element_type=_f32) + b2_ref[...]
    out_ref[...] = jnp.dot(t, tw_ref[...], preferred_element_type=_f32) + tb_ref[...]


def _head_call(g_lo, g_hi, w1, b1, w2, b2, tw, tb):
    return pl.pallas_call(
        _head_body,
        out_shape=_sds((B, 1)),
    )(g_lo, g_hi, w1, b1, w2, b2, tw, tb)


def kernel(lstm_out, first_notes, node_type, leaf_token, parent_idx, is_leaf,
           is_ptr, tree_id, ptr_pos, root_idx, embedding, leaf_W1, leaf_b1,
           leaf_W2, leaf_b2, node_W, node_b, ptr_W, ptr_b, ff_W1, ff_b1,
           ff_W2, ff_b2, tail_W, tail_b):
    nt = node_type.astype(_i32)
    lt = leaf_token.astype(_i32)
    pidx = parent_idx.astype(_i32)
    tid = tree_id.astype(_i32)
    pp = ptr_pos.astype(_i32)
    il = is_leaf.astype(_i32)
    ip = is_ptr.astype(_i32)

    t_n, t_l1, t_l2, t_p, fnh, w11, w21, w12, w22 = _fold_call(
        embedding, leaf_W1[:ED], leaf_W1[ED:], leaf_W2,
        leaf_b1.reshape(1, HID), leaf_b2.reshape(1, HID),
        node_W[:ED], node_W[ED:2 * ED], node_b.reshape(1, HID),
        first_notes, ptr_W[:64], ptr_b.reshape(1, HID), node_W[2 * ED:])
    lo2 = _lo2_call(lstm_out, fnh, ptr_W[64:])

    pidx2 = pidx.reshape(N // 128, 128)
    base_lo, base_hi, pre_lo, pre_hi, eff_all = _gather_kernel(
        nt, lt, pidx2, tid, pp, il, ip, nt, t_n, t_l1, t_l2, t_p, lo2)
    eff_all = eff_all.reshape(NPASS * N // 128, 128)

    base_lo8 = base_lo.reshape(_UP_ROWS, 128)
    base_hi8 = base_hi.reshape(_UP_ROWS, 128)
    pre_lo8 = pre_lo.reshape(_UP_ROWS, 128)
    pre_hi8 = pre_hi.reshape(_UP_ROWS, 128)
    h_lo, h_hi = base_lo, base_hi
    for _ in range(8):
        cs_lo, cs_hi = _scatter_kernel(h_lo, h_hi, eff_all)
        h_lo, h_hi = _update_call(cs_lo, cs_hi, base_lo8, base_hi8,
                                  pre_lo8, pre_hi8, w11, w21, w12, w22)

    g_lo, g_hi = _root_kernel(root_idx.astype(_i32).reshape(B // 128, 128),
                              h_lo, h_hi)
    return _head_call(g_lo, g_hi, ff_W1, ff_b1.reshape(1, HID),
                      ff_W2, ff_b2.reshape(1, HID), tail_W,
                      tail_b.reshape(1, 1))


# speedup vs baseline: 7.1076x; 1.0019x over previous
import functools

import jax
import jax.numpy as jnp
from jax import lax
from jax.experimental import pallas as pl
from jax.experimental.pallas import tpu as pltpu
from jax.experimental.pallas import tpu_sc as plsc

N = 131072
B = 256
T = 256
HID = 32
ED = 16
DICT = 200
TPAD = 208
LO_ROWS = B * T
LO_SPREAD = 2048
LO_PAD = LO_ROWS + LO_SPREAD
BIG = 1e30

NC = 2
NS = 16
NW = NC * NS
NPW = N // NW
GK = 1024
SK = 2048
ZB = 2048
PASS = 44032
NPASS = 3
TRASH = 512

_f32 = jnp.float32
_i32 = jnp.int32


def _sds(shape, dtype=_f32):
    return jax.ShapeDtypeStruct(shape, dtype)


def _fold_body(emb_ref, lw1a_ref, lw1b_ref, lw2_ref, lb1_ref, lb2_ref,
               nwp_ref, nwn_ref, nb_ref, fn_ref, pwf_ref, pb_ref, wc_ref,
               tn_ref, tl1_ref, tl2_ref, tp_ref, fnh_ref,
               w11_ref, w21_ref, w12_ref, w22_ref):
    emb = emb_ref[...]
    sel = (lax.broadcasted_iota(_i32, (128, 16), 0) % 16 ==
           lax.broadcasted_iota(_i32, (128, 16), 1)).astype(_f32)
    selt = (lax.broadcasted_iota(_i32, (16, 128), 0) ==
            lax.broadcasted_iota(_i32, (16, 128), 1) % 16).astype(_f32)
    blk = (lax.broadcasted_iota(_i32, (128, 128), 0) // 16 ==
           lax.broadcasted_iota(_i32, (128, 128), 1) // 16).astype(_f32)
    wc = wc_ref[...]

    def packw(w16):
        t = jnp.dot(jnp.dot(sel, w16, preferred_element_type=_f32), selt,
                    preferred_element_type=_f32)
        return t * blk

    w11_ref[...] = packw(wc[:16, :16])
    w21_ref[...] = packw(wc[16:, :16])
    w12_ref[...] = packw(wc[:16, 16:])
    w22_ref[...] = packw(wc[16:, 16:])
    zpad = jnp.zeros((TPAD - DICT, HID), _f32)
    tn = jnp.dot(emb, nwn_ref[...], preferred_element_type=_f32)
    tn_ref[...] = jnp.concatenate([tn, zpad], axis=0)
    a1 = jnp.dot(lw1a_ref[...], lw2_ref[...], preferred_element_type=_f32)
    tl1 = jnp.dot(emb, a1, preferred_element_type=_f32)
    tl1_ref[...] = jnp.concatenate([tl1, zpad], axis=0)
    a2 = jnp.dot(lw1b_ref[...], lw2_ref[...], preferred_element_type=_f32)
    cl = jnp.dot(lb1_ref[...], lw2_ref[...], preferred_element_type=_f32) + lb2_ref[...]
    tl2 = jnp.dot(emb, a2, preferred_element_type=_f32) + cl
    tl2_ref[...] = jnp.concatenate([tl2, zpad], axis=0)
    tp = jnp.dot(emb, nwp_ref[...], preferred_element_type=_f32) + nb_ref[...]
    ridx = lax.broadcasted_iota(_i32, (TPAD - DICT, HID), 0)
    neg = jnp.where(ridx == 1, _f32(-BIG), _f32(0.0))
    tp_ref[...] = jnp.concatenate([tp, neg], axis=0)
    fnh_ref[...] = jnp.dot(fn_ref[...], pwf_ref[...], preferred_element_type=_f32) + pb_ref[...]


def _fold_call(emb, lw1a, lw1b, lw2, lb1, lb2, nwp, nwn, nb, fn, pwf, pb, wc):
    return pl.pallas_call(
        _fold_body,
        out_shape=[_sds((TPAD, HID)), _sds((TPAD, HID)), _sds((TPAD, HID)),
                   _sds((TPAD, HID)), _sds((B, HID)),
                   _sds((128, 128)), _sds((128, 128)), _sds((128, 128)),
                   _sds((128, 128))],
    )(emb, lw1a, lw1b, lw2, lb1, lb2, nwp, nwn, nb, fn, pwf, pb, wc)


_LO_TREES = 8
_LO_BL = _LO_TREES * T
_LO_STEPS = LO_PAD // _LO_BL


def _lo2_body(lstm_ref, fnh_ref, pwl_ref, out_ref):
    i = pl.program_id(0)
    x = lstm_ref[...].reshape(_LO_BL, 128)
    y = jnp.dot(x, pwl_ref[...], preferred_element_type=_f32)
    y = (y.reshape(_LO_TREES, T, HID) + fnh_ref[...][:, None, :]).reshape(_LO_BL, HID)
    out_ref[...] = jnp.where(i >= LO_ROWS // _LO_BL, _f32(0.0), y)


def _lo2_call(lstm_out, fnh, pwl):
    return pl.pallas_call(
        _lo2_body,
        grid=(_LO_STEPS,),
        in_specs=[
            pl.BlockSpec((_LO_TREES, T, 128), lambda i: (jnp.minimum(i, B // _LO_TREES - 1), 0, 0)),
            pl.BlockSpec((_LO_TREES, HID), lambda i: (jnp.minimum(i, B // _LO_TREES - 1), 0)),
            pl.BlockSpec((128, HID), lambda i: (0, 0)),
        ],
        out_specs=pl.BlockSpec((_LO_BL, HID), lambda i: (i, 0)),
        out_shape=_sds((LO_PAD, HID)),
    )(lstm_out, fnh, pwl)


_SC_MESH = plsc.VectorSubcoreMesh(core_axis_name="c", subcore_axis_name="s")


@functools.partial(
    pl.kernel,
    out_type=[_sds((N, 16)), _sds((N, 16)), _sds((N, 16)), _sds((N, 16)),
              _sds((NPASS * N,), _i32)],
    mesh=_SC_MESH,
    compiler_params=pltpu.CompilerParams(needs_layout_passes=False, use_tc_tiling_on_sc=False),
    scratch_types=[
        pltpu.VMEM((TPAD, HID), _f32),
        pltpu.VMEM((TPAD, HID), _f32),
        pltpu.VMEM((TPAD, HID), _f32),
        pltpu.VMEM((TPAD, HID), _f32),
        pltpu.VMEM((GK,), _i32),
        pltpu.VMEM((GK,), _i32),
        pltpu.VMEM((GK,), _i32),
        pltpu.VMEM((GK,), _i32),
        pltpu.VMEM((GK,), _i32),
        pltpu.VMEM((GK,), _i32),
        pltpu.VMEM((GK // 128, 128), _i32),
        pltpu.VMEM((GK // 128, 128), _i32),
        pltpu.VMEM((GK // 128, 128), _i32),
        pltpu.VMEM((GK, HID), _f32),
        pltpu.VMEM((GK, HID), _f32),
        pltpu.VMEM((GK,), _i32),
        pltpu.VMEM((GK,), _i32),
        pltpu.VMEM((GK,), _i32),
        pltpu.SemaphoreType.DMA,
    ],
)
def _gather_kernel(nt_hbm, lt_hbm, pidx2_hbm, tid_hbm, pp_hbm, il_hbm, ip_hbm,
                   nt1_hbm, tn_hbm, tl1_hbm, tl2_hbm, tp_hbm, lo2_hbm,
                   base_lo_hbm, base_hi_hbm, pre_lo_hbm, pre_hi_hbm,
                   eff_all_hbm,
                   tn_v, tl1_v, tl2_v, tp_v,
                   nt_v, lt_v, tid_v, pp_v, il_v, ip_v,
                   pidx_v, ntp_v, flat_v, lo_v, pre_v,
                   e0_v, e1_v, e2_v, sem):
    wid = lax.axis_index("s") * NC + lax.axis_index("c")
    pltpu.sync_copy(tn_hbm, tn_v)
    pltpu.sync_copy(tl1_hbm, tl1_v)
    pltpu.sync_copy(tl2_hbm, tl2_v)
    pltpu.sync_copy(tp_hbm, tp_v)
    iota16 = lax.iota(_i32, 16)
    pending_st = []
    for chunk in range(NPW // GK):
        node0 = pl.multiple_of(wid * NPW + chunk * GK, GK)
        cps = [
            pltpu.async_copy(nt_hbm.at[pl.ds(node0, GK)], nt_v, sem),
            pltpu.async_copy(lt_hbm.at[pl.ds(node0, GK)], lt_v, sem),
            pltpu.async_copy(tid_hbm.at[pl.ds(node0, GK)], tid_v, sem),
            pltpu.async_copy(pp_hbm.at[pl.ds(node0, GK)], pp_v, sem),
            pltpu.async_copy(il_hbm.at[pl.ds(node0, GK)], il_v, sem),
            pltpu.async_copy(ip_hbm.at[pl.ds(node0, GK)], ip_v, sem),
            pltpu.async_copy(pidx2_hbm.at[pl.ds(pl.multiple_of(node0 // 128, 8), GK // 128)], pidx_v, sem),
        ]
        for cp in cps:
            cp.wait()
        for cp in pending_st:
            cp.wait()
        pending_st = []

        def pass_a(g, carry):
            r = g // 8
            s0 = (g % 8) * 16
            tid = tid_v[pl.ds(g * 16, 16)]
            pp = pp_v[pl.ds(g * 16, 16)]
            il = il_v[pl.ds(g * 16, 16)]
            ip = ip_v[pl.ds(g * 16, 16)]
            mpb = il * ip
            dump = LO_ROWS + ((g * 16) % LO_SPREAD) + iota16
            flat = jnp.where(mpb == 1, tid * T + pp, dump)
            flat_v[r, pl.ds(s0, 16)] = flat
            x = pidx_v[r, pl.ds(s0, 16)]
            tr = PASS + jnp.bitwise_and(x, TRASH - 1)
            for pno, ev in ((0, e0_v), (1, e1_v), (2, e2_v)):
                lo_b = pno * PASS
                vhi = min(PASS, N - lo_b)
                inb = jnp.logical_and(x >= lo_b, x < lo_b + vhi)
                ev[pl.ds(g * 16, 16)] = jnp.where(inb, x - lo_b, tr)
            return carry

        lax.fori_loop(0, GK // 16, pass_a, 0)

        gcps = []
        for j in range(GK // 128):
            gcps.append(pltpu.async_copy(nt1_hbm.at[pidx_v.at[j]],
                                         ntp_v.at[j], sem))
            gcps.append(pltpu.async_copy(lo2_hbm.at[flat_v.at[j]],
                                         lo_v.at[pl.ds(j * 128, 128)], sem))
        for cp in gcps:
            cp.wait()

        def pass_b(g, carry):
            r = g // 8
            s0 = (g % 8) * 16
            nt = nt_v[pl.ds(g * 16, 16)]
            lt = lt_v[pl.ds(g * 16, 16)]
            il = il_v[pl.ds(g * 16, 16)]
            ip = ip_v[pl.ds(g * 16, 16)]
            ntp = ntp_v[r, pl.ds(s0, 16)]
            mlb = il * (1 - ip)
            eff_l1 = jnp.where(mlb == 1, nt, DICT)
            eff_l2 = jnp.where(mlb == 1, lt, DICT)
            eff_p = jnp.where(il == 1, DICT + 1, ntp)
            for lane in range(16):
                node = g * 16 + lane
                l1 = eff_l1[lane]
                l2 = eff_l2[lane]
                pn = nt[lane]
                pr = eff_p[lane]
                for h in (0, 16):
                    b = tl1_v[l1, pl.ds(h, 16)] + tl2_v[l2, pl.ds(h, 16)]
                    lo_v[node, pl.ds(h, 16)] = lo_v[node, pl.ds(h, 16)] + b
                    pre_v[node, pl.ds(h, 16)] = (tn_v[pn, pl.ds(h, 16)] +
                                                 tp_v[pr, pl.ds(h, 16)])
            return carry

        lax.fori_loop(0, GK // 16, pass_b, 0)

        pending_st = [
            pltpu.async_copy(lo_v.at[:, pl.ds(0, 16)],
                             base_lo_hbm.at[pl.ds(node0, GK)], sem),
            pltpu.async_copy(lo_v.at[:, pl.ds(16, 16)],
                             base_hi_hbm.at[pl.ds(node0, GK)], sem),
            pltpu.async_copy(pre_v.at[:, pl.ds(0, 16)],
                             pre_lo_hbm.at[pl.ds(node0, GK)], sem),
            pltpu.async_copy(pre_v.at[:, pl.ds(16, 16)],
                             pre_hi_hbm.at[pl.ds(node0, GK)], sem),
            pltpu.async_copy(e0_v, eff_all_hbm.at[pl.ds(node0, GK)], sem),
            pltpu.async_copy(e1_v, eff_all_hbm.at[pl.ds(N + node0, GK)], sem),
            pltpu.async_copy(e2_v, eff_all_hbm.at[pl.ds(2 * N + node0, GK)], sem),
        ]
    for cp in pending_st:
        cp.wait()


@functools.partial(
    pl.kernel,
    out_type=[_sds((N, 16)), _sds((N, 16))],
    mesh=_SC_MESH,
    compiler_params=pltpu.CompilerParams(needs_layout_passes=False, use_tc_tiling_on_sc=False),
    scratch_types=[
        pltpu.VMEM((SK, 16), _f32),
        pltpu.VMEM((SK // 128, 128), _i32),
        pltpu.VMEM((ZB, 16), _f32),
        pltpu.VMEM_SHARED((PASS + TRASH, 16), _f32),
        pltpu.SemaphoreType.DMA,
    ],
)
def _scatter_kernel(h_lo_hbm, h_hi_hbm, eff_all_hbm,
                    cs_lo_hbm, cs_hi_hbm,
                    upd_v, eff_v, zb_v, acc, sem):
    cid = lax.axis_index("c")
    sid = lax.axis_index("s")
    z16 = jnp.zeros((16,), _f32)

    def zero_body(i, carry):
        zb_v[i, :] = z16
        return carry

    lax.fori_loop(0, ZB, zero_body, 0)
    nchunks = N // NS // SK

    def run_core(h_hbm, out_hbm):
        for p in range(NPASS):
            lo_bound = p * PASS
            valid = min(PASS, N - p * PASS)
            rpt = (PASS + TRASH) // NS
            r0 = pl.multiple_of(sid * rpt, 8)
            pltpu.sync_copy(zb_v, acc.at[pl.ds(r0, ZB)])
            pltpu.sync_copy(zb_v.at[pl.ds(0, rpt - ZB)],
                            acc.at[pl.ds(r0 + ZB, rpt - ZB)])
            plsc.subcore_barrier()

            for chunk in range(nchunks):
                src0 = pl.multiple_of(sid * (N // NS) + chunk * SK, SK)
                cp1 = pltpu.async_copy(h_hbm.at[pl.ds(src0, SK)], upd_v, sem)
                cp2 = pltpu.async_copy(
                    eff_all_hbm.at[pl.ds(pl.multiple_of(
                        p * (N // 128) + src0 // 128, 8),
                        SK // 128)], eff_v, sem)
                cp1.wait()
                cp2.wait()

                scs = [pltpu.async_copy(upd_v.at[pl.ds(j * 128, 128)],
                                        acc.at[eff_v.at[j]], sem, add=True)
                       for j in range(SK // 128)]
                for cp in scs:
                    cp.wait()
            plsc.subcore_barrier()
            wb = valid // NS
            wb0 = pl.multiple_of(sid * wb, 8)
            pltpu.sync_copy(acc.at[pl.ds(wb0, wb)],
                            out_hbm.at[pl.ds(lo_bound + wb0, wb)])
            plsc.subcore_barrier()

    @pl.when(cid == 0)
    def _():
        run_core(h_lo_hbm, cs_lo_hbm)

    @pl.when(cid == 1)
    def _():
        run_core(h_hi_hbm, cs_hi_hbm)


@functools.partial(
    pl.kernel,
    out_type=[_sds((B, 16)), _sds((B, 16))],
    mesh=_SC_MESH,
    compiler_params=pltpu.CompilerParams(needs_layout_passes=False, use_tc_tiling_on_sc=False),
    scratch_types=[
        pltpu.VMEM((B // 128, 128), _i32),
        pltpu.VMEM((B, 16), _f32),
        pltpu.VMEM((B, 16), _f32),
        pltpu.SemaphoreType.DMA,
    ],
)
def _root_kernel(ridx2_hbm, h_lo_hbm, h_hi_hbm, g_lo_hbm, g_hi_hbm,
                 idx_v, lo_v, hi_v, sem):
    wid = lax.axis_index("s") * NC + lax.axis_index("c")

    @pl.when(wid == 0)
    def _():
        pltpu.sync_copy(ridx2_hbm, idx_v)
        for j in range(B // 128):
            pltpu.async_copy(h_lo_hbm.at[idx_v.at[j]],
                             lo_v.at[pl.ds(j * 128, 128)], sem).wait()
            pltpu.async_copy(h_hi_hbm.at[idx_v.at[j]],
                             hi_v.at[pl.ds(j * 128, 128)], sem).wait()
        pltpu.sync_copy(lo_v, g_lo_hbm)
        pltpu.sync_copy(hi_v, g_hi_hbm)


_UP_ROWS = N // 8
_UP_BL = 2048


def _update_body(cs_lo_ref, cs_hi_ref, base_lo_ref, base_hi_ref,
                 pre_lo_ref, pre_hi_ref, w11_ref, w21_ref, w12_ref, w22_ref,
                 h_lo_ref, h_hi_ref):
    cl = cs_lo_ref[...]
    ch = cs_hi_ref[...]
    t_lo = jnp.maximum(
        pre_lo_ref[...] +
        jnp.dot(cl, w11_ref[...], preferred_element_type=_f32) +
        jnp.dot(ch, w21_ref[...], preferred_element_type=_f32), _f32(0.0))
    t_hi = jnp.maximum(
        pre_hi_ref[...] +
        jnp.dot(cl, w12_ref[...], preferred_element_type=_f32) +
        jnp.dot(ch, w22_ref[...], preferred_element_type=_f32), _f32(0.0))
    h_lo_ref[...] = base_lo_ref[...] + t_lo
    h_hi_ref[...] = base_hi_ref[...] + t_hi


def _update_call(cs_lo, cs_hi, base_lo8, base_hi8, pre_lo8, pre_hi8,
                 w11, w21, w12, w22):
    bspec = pl.BlockSpec((_UP_BL, 128), lambda i: (i, 0))
    wspec = pl.BlockSpec((128, 128), lambda i: (0, 0))
    h_lo8, h_hi8 = pl.pallas_call(
        _update_body,
        grid=(_UP_ROWS // _UP_BL,),
        in_specs=[bspec, bspec, bspec, bspec, bspec, bspec,
                  wspec, wspec, wspec, wspec],
        out_specs=[bspec, bspec],
        out_shape=[_sds((_UP_ROWS, 128)), _sds((_UP_ROWS, 128))],
    )(cs_lo.reshape(_UP_ROWS, 128), cs_hi.reshape(_UP_ROWS, 128),
      base_lo8, base_hi8, pre_lo8, pre_hi8, w11, w21, w12, w22)
    return h_lo8.reshape(N, 16), h_hi8.reshape(N, 16)


def _head_body(g_lo_ref, g_hi_ref, w1_ref, b1_ref, w2_ref, b2_ref,
               tw_ref, tb_ref, out_ref):
    g = jnp.concatenate([g_lo_ref[...], g_hi_ref[...]], axis=1)
    t = jnp.dot(g, w1_ref[...], preferred_element_type=_f32) + b1_ref[...]
    t = jnp.dot(t, w2_ref[...], preferred_element_type=_f32) + b2_ref[...]
    out_ref[...] = jnp.dot(t, tw_ref[...], preferred_element_type=_f32) + tb_ref[...]


def _head_call(g_lo, g_hi, w1, b1, w2, b2, tw, tb):
    return pl.pallas_call(
        _head_body,
        out_shape=_sds((B, 1)),
    )(g_lo, g_hi, w1, b1, w2, b2, tw, tb)


def kernel(lstm_out, first_notes, node_type, leaf_token, parent_idx, is_leaf,
           is_ptr, tree_id, ptr_pos, root_idx, embedding, leaf_W1, leaf_b1,
           leaf_W2, leaf_b2, node_W, node_b, ptr_W, ptr_b, ff_W1, ff_b1,
           ff_W2, ff_b2, tail_W, tail_b):
    nt = node_type.astype(_i32)
    lt = leaf_token.astype(_i32)
    pidx = parent_idx.astype(_i32)
    tid = tree_id.astype(_i32)
    pp = ptr_pos.astype(_i32)
    il = is_leaf.astype(_i32)
    ip = is_ptr.astype(_i32)

    t_n, t_l1, t_l2, t_p, fnh, w11, w21, w12, w22 = _fold_call(
        embedding, leaf_W1[:ED], leaf_W1[ED:], leaf_W2,
        leaf_b1.reshape(1, HID), leaf_b2.reshape(1, HID),
        node_W[:ED], node_W[ED:2 * ED], node_b.reshape(1, HID),
        first_notes, ptr_W[:64], ptr_b.reshape(1, HID), node_W[2 * ED:])
    lo2 = _lo2_call(lstm_out, fnh, ptr_W[64:])

    pidx2 = pidx.reshape(N // 128, 128)
    base_lo, base_hi, pre_lo, pre_hi, eff_all = _gather_kernel(
        nt, lt, pidx2, tid, pp, il, ip, nt, t_n, t_l1, t_l2, t_p, lo2)
    eff_all = eff_all.reshape(NPASS * N // 128, 128)

    base_lo8 = base_lo.reshape(_UP_ROWS, 128)
    base_hi8 = base_hi.reshape(_UP_ROWS, 128)
    pre_lo8 = pre_lo.reshape(_UP_ROWS, 128)
    pre_hi8 = pre_hi.reshape(_UP_ROWS, 128)
    h_lo, h_hi = base_lo, base_hi
    for _ in range(8):
        cs_lo, cs_hi = _scatter_kernel(h_lo, h_hi, eff_all)
        h_lo, h_hi = _update_call(cs_lo, cs_hi, base_lo8, base_hi8,
                                  pre_lo8, pre_hi8, w11, w21, w12, w22)

    g_lo, g_hi = _root_kernel(root_idx.astype(_i32).reshape(B // 128, 128),
                              h_lo, h_hi)
    return _head_call(g_lo, g_hi, ff_W1, ff_b1.reshape(1, HID),
                      ff_W2, ff_b2.reshape(1, HID), tail_W,
                      tail_b.reshape(1, 1))
